# baseline (speedup 1.0000x reference)
import functools

import numpy as _np

import jax
import jax.numpy as jnp
from jax import lax
from jax.experimental import pallas as pl
from jax.experimental.pallas import tpu as pltpu
from jax.experimental.pallas import tpu_sc as plsc

F32 = jnp.float32

N = 10000
E = 320000
DI = 128
H2 = 128
HID = 256

NSUB = 16
NCORE = 2
BG = 80
BC = 120
RPT = 640
NP = NSUB * RPT

DW = DI + 16
WI = DI // 2 + 8

_base = _np.arange(0, 32, 2)
_blk = _np.concatenate([_base, _base + 1])
PERM = _np.concatenate([_blk + 32 * k for k in range(4)])
PERM2 = PERM[PERM]


def _even_blocks(edges, be):
    nb = -(-edges // (NSUB * be))
    return nb + (nb % 2)


EG = E + N
NBG = _even_blocks(EG, BG)
NBSA = _even_blocks(E, BG)
NBC = _even_blocks(E, BC)
EGP = NSUB * NBG * BG
ESPA = NSUB * NBSA * BG
ESPC = NSUB * NBC * BC

RB = 512
GRID = NP // RB


def _dot(a, b):
    return jnp.dot(a, b, preferred_element_type=F32)


def _tc_pre_body(x_ref, wg_ref, asrc_ref, adst_ref,
                 xw_ref, as_ref, ad_ref):
    x = x_ref[...]
    xw = _dot(x, wg_ref[...])
    xw_ref[...] = xw
    as_ref[...] = _dot(xw, asrc_ref[...])
    ad_ref[...] = _dot(xw, adst_ref[...])


def _tc_pre(xP, W_gat, a_src, a_dst):
    row = lambda i: (i, 0)
    full = lambda i: (0, 0)
    return pl.pallas_call(
        _tc_pre_body,
        grid=(GRID,),
        in_specs=[
            pl.BlockSpec((RB, DI), row),
            pl.BlockSpec((DI, H2), full),
            pl.BlockSpec((H2, 1), full),
            pl.BlockSpec((H2, 1), full),
        ],
        out_specs=[
            pl.BlockSpec((RB, DI), row),
            pl.BlockSpec((RB, 1), row),
            pl.BlockSpec((RB, 1), row),
        ],
        out_shape=[
            jax.ShapeDtypeStruct((NP, DI), F32),
            jax.ShapeDtypeStruct((NP, 1), F32),
            jax.ShapeDtypeStruct((NP, 1), F32),
        ],
    )(xP, W_gat, a_src, a_dst)


def _pack_words(a):
    ab = a.astype(jnp.bfloat16)
    return lax.bitcast_convert_type(
        ab.reshape(a.shape[0], a.shape[1] // 2, 2), jnp.int32)


def _layernorm(h, g, b):
    mu = jnp.mean(h, axis=1, keepdims=True)
    d = h - mu
    var = jnp.mean(d * d, axis=1, keepdims=True)
    return d * lax.rsqrt(var + 1e-5) * g + b


def _tc_mid_body(md_ref, ag_ref, x_ref, bgat_ref, w1l_ref, b1l_ref,
                 w1r_ref, g1_ref, be1_ref,
                 hlo_ref, hhi_ref):
    md = md_ref[...]
    ag = ag_ref[...]
    x1 = md[:, :H2] / jnp.maximum(md[:, H2:H2 + 1], 1e-16) + bgat_ref[...]
    mean1 = ag[:, :DI] / jnp.maximum(ag[:, DI:DI + 1], 1.0)
    x2 = _dot(mean1, w1l_ref[...]) + b1l_ref[...] + _dot(x_ref[...], w1r_ref[...])
    h = jnp.concatenate([x1, x2], axis=1)
    h = _layernorm(h, g1_ref[...], be1_ref[...])
    h = jnp.maximum(h, 0.0)
    hlo_ref[...] = h[:, :H2]
    hhi_ref[...] = h[:, H2:]


def _tc_mid(md, ag, xP, bgat, W1_l, b1l, W1_r, g1, be1):
    row = lambda i: (i, 0)
    full = lambda i: (0, 0)
    return pl.pallas_call(
        _tc_mid_body,
        grid=(GRID,),
        in_specs=[
            pl.BlockSpec((RB, DW), row),
            pl.BlockSpec((RB, DW), row),
            pl.BlockSpec((RB, DI), row),
            pl.BlockSpec((1, H2), full),
            pl.BlockSpec((DI, H2), full),
            pl.BlockSpec((1, H2), full),
            pl.BlockSpec((DI, H2), full),
            pl.BlockSpec((1, HID), full),
            pl.BlockSpec((1, HID), full),
        ],
        out_specs=[
            pl.BlockSpec((RB, H2), row),
            pl.BlockSpec((RB, H2), row),
        ],
        out_shape=[
            jax.ShapeDtypeStruct((NP, H2), F32),
            jax.ShapeDtypeStruct((NP, H2), F32),
        ],
    )(md, ag, xP, bgat, W1_l, b1l, W1_r, g1, be1)


def _tc_hr_body(hlo_ref, hhi_ref, w2r_ref, b2l_ref, hr_ref):
    w2r = w2r_ref[...]
    hr_ref[...] = (_dot(hlo_ref[...], w2r[:H2, :])
                   + _dot(hhi_ref[...], w2r[H2:, :]) + b2l_ref[...])


def _tc_hr(hlo, hhi, W2_r, b2l):
    row = lambda i: (i, 0)
    full = lambda i: (0, 0)
    return pl.pallas_call(
        _tc_hr_body,
        grid=(GRID,),
        in_specs=[
            pl.BlockSpec((RB, H2), row),
            pl.BlockSpec((RB, H2), row),
            pl.BlockSpec((HID, HID), full),
            pl.BlockSpec((1, HID), full),
        ],
        out_specs=pl.BlockSpec((RB, HID), row),
        out_shape=jax.ShapeDtypeStruct((NP, HID), F32),
    )(hlo, hhi, W2_r, b2l)


def _tc_fin_body(alo_ref, ahi_ref, deg_ref, hr_ref, w2l_ref,
                 g2_ref, be2_ref, wc_ref, bc_ref, out_ref):
    deg = jnp.maximum(deg_ref[...], 1.0)
    m2l = alo_ref[...] / deg
    m2h = ahi_ref[...] / deg
    w2l = w2l_ref[...]
    h2 = (_dot(m2l, w2l[:H2, :]) + _dot(m2h, w2l[H2:, :])
          + hr_ref[...])
    h2 = _layernorm(h2, g2_ref[...], be2_ref[...])
    h2 = jnp.maximum(h2, 0.0)
    out_ref[...] = _dot(h2, wc_ref[...]) + bc_ref[...]


def _tc_fin(alo, ahi, deg, hr, W2_l, g2, be2, Wc, bc):
    row = lambda i: (i, 0)
    full = lambda i: (0, 0)
    return pl.pallas_call(
        _tc_fin_body,
        grid=(GRID,),
        in_specs=[
            pl.BlockSpec((RB, H2), row),
            pl.BlockSpec((RB, H2), row),
            pl.BlockSpec((RB, 1), row),
            pl.BlockSpec((RB, HID), row),
            pl.BlockSpec((HID, HID), full),
            pl.BlockSpec((1, HID), full),
            pl.BlockSpec((1, HID), full),
            pl.BlockSpec((HID, 1), full),
            pl.BlockSpec((1, 1), full),
        ],
        out_specs=pl.BlockSpec((RB, 1), row),
        out_shape=jax.ShapeDtypeStruct((NP, 1), F32),
    )(alo, ahi, deg, hr, W2_l, g2, be2, Wc, bc)


def _zero_acc(r0, acc, width, sid):
    zv = jnp.zeros((16,), F32)
    for r in range(16):
        for k in range(width // 16):
            r0[r, pl.ds(k * 16, 16)] = zv
    base = sid * RPT

    def body(i, _):
        off = pl.multiple_of(base + i * 16, 16)
        pltpu.sync_copy(r0.at[pl.ds(0, 16)], acc.at[pl.ds(off, 16)])
        return 0

    lax.fori_loop(0, RPT // 16, body, 0)


NAT_FIRST = True


def _unpack32(v):
    a, b = plsc.unpack(plsc.bitcast(v, jnp.bfloat16),
                       format=plsc.PackFormat.INTERLEAVED)
    return (a, b) if NAT_FIRST else (b, a)


def _pipe_plain(nblk, sid, src2, dst2, table, bufs, acc, width):
    (s0, d0, b0r, f0, g0, ss0), (s1, d1, b1r, f1, g1, ss1) = bufs
    half = nblk // 2
    nb = src2.shape[1]
    wcols = width // 32

    def fire(bid, sidx, didx, gsem, braw):
        pltpu.sync_copy(src2.at[bid], sidx)
        pltpu.sync_copy(dst2.at[bid], didx)
        pltpu.async_copy(table.at[sidx], braw, gsem)

    def process(sidx, didx, gsem, braw, fbuf, ssem):
        pltpu.make_async_copy(table.at[sidx], braw, gsem).wait()
        for j in range(nb):
            for k in range(wcols):
                fa, fb = _unpack32(braw[j, pl.ds(k * 16, 16)])
                fbuf[j, pl.ds(k * 32, 16)] = fa
                fbuf[j, pl.ds(k * 32 + 16, 16)] = fb
        pltpu.async_copy(fbuf, acc.at[didx], ssem, add=True)

    fire(sid * nblk, s0, d0, g0, b0r)
    fire(sid * nblk + 1, s1, d1, g1, b1r)

    def body(i, _):
        bb = sid * nblk + 2 * i
        process(s0, d0, g0, b0r, f0, ss0)
        process(s1, d1, g1, b1r, f1, ss1)

        @pl.when(i < half - 1)
        def _():
            pltpu.make_async_copy(f0, acc.at[d0], ss0).wait()
            fire(bb + 2, s0, d0, g0, b0r)
            pltpu.make_async_copy(f1, acc.at[d1], ss1).wait()
            fire(bb + 3, s1, d1, g1, b1r)

        return 0

    lax.fori_loop(0, half, body, 0)
    pltpu.make_async_copy(f0, acc.at[d0], ss0).wait()
    pltpu.make_async_copy(f1, acc.at[d1], ss1).wait()


def _pipe_gat(sid, src2, dst2, xwt, adt, bufs, acc):
    (s0, d0, b0r, a0, f0, g0, ss0), (s1, d1, b1r, a1, f1, g1, ss1) = bufs
    half = NBG // 2
    lane = lax.iota(jnp.int32, 16)
    as_col = jnp.full((16,), DI // 2, jnp.int32)
    zero_col = jnp.zeros((16,), jnp.int32)

    def fire(bid, sidx, didx, gsem, braw, adr):
        pltpu.sync_copy(src2.at[bid], sidx)
        pltpu.sync_copy(dst2.at[bid], didx)
        pltpu.async_copy(xwt.at[sidx], braw, gsem)
        pltpu.async_copy(adt.at[didx], adr, gsem)

    def process(sidx, didx, gsem, braw, adr, fbuf, ssem):
        pltpu.make_async_copy(xwt.at[sidx], braw, gsem).wait()
        pltpu.make_async_copy(adt.at[didx], adr, gsem).wait()
        for j8 in range(BG // 16):
            jvec = lane + (j8 * 16)
            as_f, _ = _unpack32(plsc.load_gather(braw, [jvec, as_col]))
            ad_f = plsc.load_gather(adr, [jvec, zero_col])
            u = as_f + ad_f
            wv = jnp.exp(jnp.maximum(u, 0.2 * u))
            for l in range(16):
                w = wv[l]
                j = j8 * 16 + l
                for k in range(DI // 32):
                    fa, fb = _unpack32(braw[j, pl.ds(k * 16, 16)])
                    fbuf[j, pl.ds(k * 32, 16)] = fa * w
                    fbuf[j, pl.ds(k * 32 + 16, 16)] = fb * w
                fbuf[j, pl.ds(DI, 16)] = lax.broadcast(w, (16,))
        pltpu.async_copy(fbuf, acc.at[didx], ssem, add=True)

    fire(sid * NBG, s0, d0, g0, b0r, a0)
    fire(sid * NBG + 1, s1, d1, g1, b1r, a1)

    def body(i, _):
        bb = sid * NBG + 2 * i
        process(s0, d0, g0, b0r, a0, f0, ss0)
        process(s1, d1, g1, b1r, a1, f1, ss1)

        @pl.when(i < half - 1)
        def _():
            pltpu.make_async_copy(f0, acc.at[d0], ss0).wait()
            fire(bb + 2, s0, d0, g0, b0r, a0)
            pltpu.make_async_copy(f1, acc.at[d1], ss1).wait()
            fire(bb + 3, s1, d1, g1, b1r, a1)

        return 0

    lax.fori_loop(0, half, body, 0)
    pltpu.make_async_copy(f0, acc.at[d0], ss0).wait()
    pltpu.make_async_copy(f1, acc.at[d1], ss1).wait()


def _sc_stage_a_body(xwt, xt, adt, gsrc2, gdst2, ssrc2, sdst2,
                     md_out, ag_out,
                     s0, s1, d0, d1, b0r, b1r, a0, a1, f0, f1,
                     g0, g1, ss0, ss1, acc):
    cid = lax.axis_index("c")
    sid = lax.axis_index("s")

    _zero_acc(f0, acc, DW, sid)
    plsc.subcore_barrier()

    @pl.when(cid == 0)
    def _():
        _pipe_gat(sid, gsrc2, gdst2, xwt, adt,
                  ((s0, d0, b0r, a0, f0, g0, ss0),
                   (s1, d1, b1r, a1, f1, g1, ss1)),
                  acc)

    @pl.when(cid == 1)
    def _():
        tail = jnp.where(lax.iota(jnp.int32, 16) == 0, 1.0, 0.0)
        for fbuf in (f0, f1):
            for j in range(BG):
                fbuf[j, pl.ds(DI, 16)] = tail
        _pipe_plain(NBSA, sid, ssrc2, sdst2, xt,
                    ((s0, d0, b0r, f0, g0, ss0),
                     (s1, d1, b1r, f1, g1, ss1)), acc, DI)

    plsc.subcore_barrier()

    base = pl.multiple_of(sid * RPT, 16)

    @pl.when(cid == 0)
    def _():
        pltpu.sync_copy(acc.at[pl.ds(base, RPT)], md_out.at[pl.ds(base, RPT)])

    @pl.when(cid == 1)
    def _():
        pltpu.sync_copy(acc.at[pl.ds(base, RPT)], ag_out.at[pl.ds(base, RPT)])


def _sc_stage_a(xwt, xt, adt, gsrc2, gdst2, ssrc2, sdst2):
    mesh = plsc.VectorSubcoreMesh(core_axis_name="c", subcore_axis_name="s")
    return pl.kernel(
        _sc_stage_a_body,
        out_type=[
            jax.ShapeDtypeStruct((NP, DW), F32),
            jax.ShapeDtypeStruct((NP, DW), F32),
        ],
        mesh=mesh,
        compiler_params=pltpu.CompilerParams(
            needs_layout_passes=False, use_tc_tiling_on_sc=False),
        scratch_types=[
            pltpu.VMEM((BG,), jnp.int32),
            pltpu.VMEM((BG,), jnp.int32),
            pltpu.VMEM((BG,), jnp.int32),
            pltpu.VMEM((BG,), jnp.int32),
            pltpu.VMEM((BG, WI), jnp.int32),
            pltpu.VMEM((BG, WI), jnp.int32),
            pltpu.VMEM((BG, 16), F32),
            pltpu.VMEM((BG, 16), F32),
            pltpu.VMEM((BG, DW), F32),
            pltpu.VMEM((BG, DW), F32),
            pltpu.SemaphoreType.DMA,
            pltpu.SemaphoreType.DMA,
            pltpu.SemaphoreType.DMA,
            pltpu.SemaphoreType.DMA,
            pltpu.VMEM_SHARED((NP, DW), F32),
        ],
    )(xwt, xt, adt, gsrc2, gdst2, ssrc2, sdst2)


def _sc_stage_c_body(blo, bhi, ssrc2, sdst2, alo_out, ahi_out,
                     s0, s1, d0, d1, b0r, b1r, f0, f1,
                     g0, g1, ss0, ss1, acc):
    cid = lax.axis_index("c")
    sid = lax.axis_index("s")

    _zero_acc(f0, acc, H2, sid)
    plsc.subcore_barrier()

    bufs = ((s0, d0, b0r, f0, g0, ss0), (s1, d1, b1r, f1, g1, ss1))

    @pl.when(cid == 0)
    def _():
        _pipe_plain(NBC, sid, ssrc2, sdst2, blo, bufs, acc, H2)

    @pl.when(cid == 1)
    def _():
        _pipe_plain(NBC, sid, ssrc2, sdst2, bhi, bufs, acc, H2)

    plsc.subcore_barrier()

    base = pl.multiple_of(sid * RPT, 16)

    @pl.when(cid == 0)
    def _():
        pltpu.sync_copy(acc.at[pl.ds(base, RPT)], alo_out.at[pl.ds(base, RPT)])

    @pl.when(cid == 1)
    def _():
        pltpu.sync_copy(acc.at[pl.ds(base, RPT)], ahi_out.at[pl.ds(base, RPT)])


def _sc_stage_c(blo, bhi, ssrc2, sdst2):
    mesh = plsc.VectorSubcoreMesh(core_axis_name="c", subcore_axis_name="s")
    return pl.kernel(
        _sc_stage_c_body,
        out_type=[
            jax.ShapeDtypeStruct((NP, H2), F32),
            jax.ShapeDtypeStruct((NP, H2), F32),
        ],
        mesh=mesh,
        compiler_params=pltpu.CompilerParams(
            needs_layout_passes=False, use_tc_tiling_on_sc=False),
        scratch_types=[
            pltpu.VMEM((BC,), jnp.int32),
            pltpu.VMEM((BC,), jnp.int32),
            pltpu.VMEM((BC,), jnp.int32),
            pltpu.VMEM((BC,), jnp.int32),
            pltpu.VMEM((BC, H2 // 2), jnp.int32),
            pltpu.VMEM((BC, H2 // 2), jnp.int32),
            pltpu.VMEM((BC, H2), F32),
            pltpu.VMEM((BC, H2), F32),
            pltpu.SemaphoreType.DMA,
            pltpu.SemaphoreType.DMA,
            pltpu.SemaphoreType.DMA,
            pltpu.SemaphoreType.DMA,
            pltpu.VMEM_SHARED((NP, H2), F32),
        ],
    )(blo, bhi, ssrc2, sdst2)


def kernel(x, edge_index, W_gat, att_src, att_dst, b_gat, W1_l, b1_l, W1_r,
           g1, be1, W2_l, b2_l, W2_r, g2, be2, Wc, bc):
    src = edge_index[0]
    dst = edge_index[1]
    loops = jnp.arange(N, dtype=jnp.int32)

    gsrc = jnp.concatenate([src, loops])
    gdst = jnp.concatenate([dst, loops])
    gsrc2 = jnp.pad(gsrc, (0, EGP - EG)).reshape(NSUB * NBG, BG)
    gdst2 = jnp.pad(gdst, (0, EGP - EG), constant_values=N).reshape(NSUB * NBG, BG)
    ssrcA = jnp.pad(src, (0, ESPA - E)).reshape(NSUB * NBSA, BG)
    sdstA = jnp.pad(dst, (0, ESPA - E), constant_values=N).reshape(NSUB * NBSA, BG)
    ssrcC = jnp.pad(src, (0, ESPC - E)).reshape(NSUB * NBC, BC)
    sdstC = jnp.pad(dst, (0, ESPC - E), constant_values=N).reshape(NSUB * NBC, BC)

    xP = jnp.pad(x, ((0, NP - N), (0, 0)))

    xw, as2, ad2 = _tc_pre(xP, W_gat,
                           att_src.reshape(H2, 1), att_dst.reshape(H2, 1))

    zpad = jnp.zeros((NP, WI - DI // 2 - 1), jnp.int32)
    asw = _pack_words(jnp.concatenate([as2, jnp.zeros((NP, 1), F32)], axis=1))
    xwt = jnp.concatenate([_pack_words(xw), asw, zpad], axis=1)
    xt = jnp.concatenate(
        [_pack_words(xP), jnp.zeros((NP, WI - DI // 2), jnp.int32)], axis=1)
    adt = jnp.concatenate([ad2, jnp.zeros((NP, 15), F32)], axis=1)

    md, ag = _sc_stage_a(xwt, xt, adt, gsrc2, gdst2, ssrcA, sdstA)

    hlo, hhi = _tc_mid(
        md, ag, xP, b_gat[PERM].reshape(1, H2), W1_l[PERM, :],
        b1_l.reshape(1, H2), W1_r,
        jnp.concatenate([g1[:H2][PERM], g1[H2:]]).reshape(1, HID),
        jnp.concatenate([be1[:H2][PERM], be1[H2:]]).reshape(1, HID))

    W2r_p = jnp.concatenate([W2_r[:H2][PERM], W2_r[H2:]], axis=0)
    hr = _tc_hr(hlo, hhi, W2r_p, b2_l.reshape(1, HID))

    alo, ahi = _sc_stage_c(_pack_words(hlo), _pack_words(hhi), ssrcC, sdstC)

    W2l_p = jnp.concatenate([W2_l[:H2][PERM2], W2_l[H2:][PERM]], axis=0)
    deg = lax.slice(ag, (0, DI), (NP, DI + 1))
    out = _tc_fin(alo, ahi, deg, hr, W2l_p,
                  g2.reshape(1, HID), be2.reshape(1, HID), Wc,
                  bc.reshape(1, 1))
    return out[:N, 0]


# speedup vs baseline: 1.0001x; 1.0001x over previous
import functools

import numpy as _np

import jax
import jax.numpy as jnp
from jax import lax
from jax.experimental import pallas as pl
from jax.experimental.pallas import tpu as pltpu
from jax.experimental.pallas import tpu_sc as plsc

F32 = jnp.float32

N = 10000
E = 320000
DI = 128
H2 = 128
HID = 256

NSUB = 16
NCORE = 2
BG = 80
BC = 120
RPT = 640
NP = NSUB * RPT

DW = DI + 16
WI = DI // 2 + 8

_base = _np.arange(0, 32, 2)
_blk = _np.concatenate([_base, _base + 1])
PERM = _np.concatenate([_blk + 32 * k for k in range(4)])
PERM2 = PERM[PERM]


def _even_blocks(edges, be):
    nb = -(-edges // (NSUB * be))
    return nb + (nb % 2)


EG = E + N
NBG = _even_blocks(EG, BG)
NBSA = _even_blocks(E, BG)
NBC = _even_blocks(E, BC)
EGP = NSUB * NBG * BG
ESPA = NSUB * NBSA * BG
ESPC = NSUB * NBC * BC

RB = 512
GRID = NP // RB


def _dot(a, b):
    return jnp.dot(a, b, preferred_element_type=F32)


def _tc_pre_body(x_ref, wg_ref, asrc_ref, adst_ref,
                 xw_ref, as_ref, ad_ref):
    x = x_ref[...]
    xw = _dot(x, wg_ref[...])
    xw_ref[...] = xw
    as_ref[...] = _dot(xw, asrc_ref[...])
    ad_ref[...] = _dot(xw, adst_ref[...])


def _tc_pre(xP, W_gat, a_src, a_dst):
    row = lambda i: (i, 0)
    full = lambda i: (0, 0)
    return pl.pallas_call(
        _tc_pre_body,
        grid=(GRID,),
        in_specs=[
            pl.BlockSpec((RB, DI), row),
            pl.BlockSpec((DI, H2), full),
            pl.BlockSpec((H2, 1), full),
            pl.BlockSpec((H2, 1), full),
        ],
        out_specs=[
            pl.BlockSpec((RB, DI), row),
            pl.BlockSpec((RB, 1), row),
            pl.BlockSpec((RB, 1), row),
        ],
        out_shape=[
            jax.ShapeDtypeStruct((NP, DI), F32),
            jax.ShapeDtypeStruct((NP, 1), F32),
            jax.ShapeDtypeStruct((NP, 1), F32),
        ],
    )(xP, W_gat, a_src, a_dst)


def _pack_words(a):
    ab = a.astype(jnp.bfloat16)
    return lax.bitcast_convert_type(
        ab.reshape(a.shape[0], a.shape[1] // 2, 2), jnp.int32)


def _layernorm(h, g, b):
    mu = jnp.mean(h, axis=1, keepdims=True)
    d = h - mu
    var = jnp.mean(d * d, axis=1, keepdims=True)
    return d * lax.rsqrt(var + 1e-5) * g + b


def _tc_mid_body(md_ref, ag_ref, x_ref, bgat_ref, w1l_ref, b1l_ref,
                 w1r_ref, g1_ref, be1_ref,
                 hlo_ref, hhi_ref):
    md = md_ref[...]
    ag = ag_ref[...]
    x1 = md[:, :H2] / jnp.maximum(md[:, H2:H2 + 1], 1e-16) + bgat_ref[...]
    mean1 = ag[:, :DI] / jnp.maximum(ag[:, DI:DI + 1], 1.0)
    x2 = _dot(mean1, w1l_ref[...]) + b1l_ref[...] + _dot(x_ref[...], w1r_ref[...])
    h = jnp.concatenate([x1, x2], axis=1)
    h = _layernorm(h, g1_ref[...], be1_ref[...])
    h = jnp.maximum(h, 0.0)
    hlo_ref[...] = h[:, :H2]
    hhi_ref[...] = h[:, H2:]


def _tc_mid(md, ag, xP, bgat, W1_l, b1l, W1_r, g1, be1):
    row = lambda i: (i, 0)
    full = lambda i: (0, 0)
    return pl.pallas_call(
        _tc_mid_body,
        grid=(GRID,),
        in_specs=[
            pl.BlockSpec((RB, DW), row),
            pl.BlockSpec((RB, DW), row),
            pl.BlockSpec((RB, DI), row),
            pl.BlockSpec((1, H2), full),
            pl.BlockSpec((DI, H2), full),
            pl.BlockSpec((1, H2), full),
            pl.BlockSpec((DI, H2), full),
            pl.BlockSpec((1, HID), full),
            pl.BlockSpec((1, HID), full),
        ],
        out_specs=[
            pl.BlockSpec((RB, H2), row),
            pl.BlockSpec((RB, H2), row),
        ],
        out_shape=[
            jax.ShapeDtypeStruct((NP, H2), F32),
            jax.ShapeDtypeStruct((NP, H2), F32),
        ],
    )(md, ag, xP, bgat, W1_l, b1l, W1_r, g1, be1)


def _tc_hr_body(hlo_ref, hhi_ref, w2r_ref, b2l_ref, hr_ref):
    w2r = w2r_ref[...]
    hr_ref[...] = (_dot(hlo_ref[...], w2r[:H2, :])
                   + _dot(hhi_ref[...], w2r[H2:, :]) + b2l_ref[...])


def _tc_hr(hlo, hhi, W2_r, b2l):
    row = lambda i: (i, 0)
    full = lambda i: (0, 0)
    return pl.pallas_call(
        _tc_hr_body,
        grid=(GRID,),
        in_specs=[
            pl.BlockSpec((RB, H2), row),
            pl.BlockSpec((RB, H2), row),
            pl.BlockSpec((HID, HID), full),
            pl.BlockSpec((1, HID), full),
        ],
        out_specs=pl.BlockSpec((RB, HID), row),
        out_shape=jax.ShapeDtypeStruct((NP, HID), F32),
    )(hlo, hhi, W2_r, b2l)


def _tc_fin_body(alo_ref, ahi_ref, deg_ref, hr_ref, w2l_ref,
                 g2_ref, be2_ref, wc_ref, bc_ref, out_ref):
    deg = jnp.maximum(deg_ref[...], 1.0)
    m2l = alo_ref[...] / deg
    m2h = ahi_ref[...] / deg
    w2l = w2l_ref[...]
    h2 = (_dot(m2l, w2l[:H2, :]) + _dot(m2h, w2l[H2:, :])
          + hr_ref[...])
    h2 = _layernorm(h2, g2_ref[...], be2_ref[...])
    h2 = jnp.maximum(h2, 0.0)
    out_ref[...] = _dot(h2, wc_ref[...]) + bc_ref[...]


def _tc_fin(alo, ahi, deg, hr, W2_l, g2, be2, Wc, bc):
    row = lambda i: (i, 0)
    full = lambda i: (0, 0)
    return pl.pallas_call(
        _tc_fin_body,
        grid=(GRID,),
        in_specs=[
            pl.BlockSpec((RB, H2), row),
            pl.BlockSpec((RB, H2), row),
            pl.BlockSpec((RB, 1), row),
            pl.BlockSpec((RB, HID), row),
            pl.BlockSpec((HID, HID), full),
            pl.BlockSpec((1, HID), full),
            pl.BlockSpec((1, HID), full),
            pl.BlockSpec((HID, 1), full),
            pl.BlockSpec((1, 1), full),
        ],
        out_specs=pl.BlockSpec((RB, 1), row),
        out_shape=jax.ShapeDtypeStruct((NP, 1), F32),
    )(alo, ahi, deg, hr, W2_l, g2, be2, Wc, bc)


def _zero_acc(r0, acc, width, sid):
    zv = jnp.zeros((16,), F32)
    for r in range(16):
        for k in range(width // 16):
            r0[r, pl.ds(k * 16, 16)] = zv
    base = sid * RPT

    def body(i, _):
        off = pl.multiple_of(base + i * 16, 16)
        pltpu.sync_copy(r0.at[pl.ds(0, 16)], acc.at[pl.ds(off, 16)])
        return 0

    lax.fori_loop(0, RPT // 16, body, 0)


def _unpack32(v):
    fa = plsc.bitcast(v << 16, F32)
    fb = plsc.bitcast(v & jnp.int32(-65536), F32)
    return fa, fb


def _pipe_plain(nblk, sid, src2, dst2, table, bufs, acc, width):
    (s0, d0, b0r, f0, g0, ss0), (s1, d1, b1r, f1, g1, ss1) = bufs
    half = nblk // 2
    nb = src2.shape[1]
    wcols = width // 32

    def fire(bid, sidx, didx, gsem, braw):
        pltpu.sync_copy(src2.at[bid], sidx)
        pltpu.sync_copy(dst2.at[bid], didx)
        pltpu.async_copy(table.at[sidx], braw, gsem)

    def process(sidx, didx, gsem, braw, fbuf, ssem):
        pltpu.make_async_copy(table.at[sidx], braw, gsem).wait()
        for j in range(nb):
            for k in range(wcols):
                fa, fb = _unpack32(braw[j, pl.ds(k * 16, 16)])
                fbuf[j, pl.ds(k * 32, 16)] = fa
                fbuf[j, pl.ds(k * 32 + 16, 16)] = fb
        pltpu.async_copy(fbuf, acc.at[didx], ssem, add=True)

    fire(sid * nblk, s0, d0, g0, b0r)
    fire(sid * nblk + 1, s1, d1, g1, b1r)

    def body(i, _):
        bb = sid * nblk + 2 * i
        process(s0, d0, g0, b0r, f0, ss0)
        process(s1, d1, g1, b1r, f1, ss1)

        @pl.when(i < half - 1)
        def _():
            pltpu.make_async_copy(f0, acc.at[d0], ss0).wait()
            fire(bb + 2, s0, d0, g0, b0r)
            pltpu.make_async_copy(f1, acc.at[d1], ss1).wait()
            fire(bb + 3, s1, d1, g1, b1r)

        return 0

    lax.fori_loop(0, half, body, 0)
    pltpu.make_async_copy(f0, acc.at[d0], ss0).wait()
    pltpu.make_async_copy(f1, acc.at[d1], ss1).wait()


def _pipe_gat(sid, src2, dst2, xwt, adt, bufs, acc):
    (s0, d0, b0r, a0, f0, g0, ss0), (s1, d1, b1r, a1, f1, g1, ss1) = bufs
    half = NBG // 2
    lane = lax.iota(jnp.int32, 16)
    as_col = jnp.full((16,), DI // 2, jnp.int32)
    zero_col = jnp.zeros((16,), jnp.int32)

    def fire(bid, sidx, didx, gsem, braw, adr):
        pltpu.sync_copy(src2.at[bid], sidx)
        pltpu.sync_copy(dst2.at[bid], didx)
        pltpu.async_copy(xwt.at[sidx], braw, gsem)
        pltpu.async_copy(adt.at[didx], adr, gsem)

    def process(sidx, didx, gsem, braw, adr, fbuf, ssem):
        pltpu.make_async_copy(xwt.at[sidx], braw, gsem).wait()
        pltpu.make_async_copy(adt.at[didx], adr, gsem).wait()
        for j8 in range(BG // 16):
            jvec = lane + (j8 * 16)
            as_f, _ = _unpack32(plsc.load_gather(braw, [jvec, as_col]))
            ad_f = plsc.load_gather(adr, [jvec, zero_col])
            u = as_f + ad_f
            wv = jnp.exp(jnp.maximum(u, 0.2 * u))
            for l in range(16):
                w = wv[l]
                j = j8 * 16 + l
                for k in range(DI // 32):
                    fa, fb = _unpack32(braw[j, pl.ds(k * 16, 16)])
                    fbuf[j, pl.ds(k * 32, 16)] = fa * w
                    fbuf[j, pl.ds(k * 32 + 16, 16)] = fb * w
                fbuf[j, pl.ds(DI, 16)] = lax.broadcast(w, (16,))
        pltpu.async_copy(fbuf, acc.at[didx], ssem, add=True)

    fire(sid * NBG, s0, d0, g0, b0r, a0)
    fire(sid * NBG + 1, s1, d1, g1, b1r, a1)

    def body(i, _):
        bb = sid * NBG + 2 * i
        process(s0, d0, g0, b0r, a0, f0, ss0)
        process(s1, d1, g1, b1r, a1, f1, ss1)

        @pl.when(i < half - 1)
        def _():
            pltpu.make_async_copy(f0, acc.at[d0], ss0).wait()
            fire(bb + 2, s0, d0, g0, b0r, a0)
            pltpu.make_async_copy(f1, acc.at[d1], ss1).wait()
            fire(bb + 3, s1, d1, g1, b1r, a1)

        return 0

    lax.fori_loop(0, half, body, 0)
    pltpu.make_async_copy(f0, acc.at[d0], ss0).wait()
    pltpu.make_async_copy(f1, acc.at[d1], ss1).wait()


def _sc_stage_a_body(xwt, xt, adt, gsrc2, gdst2, ssrc2, sdst2,
                     md_out, ag_out,
                     s0, s1, d0, d1, b0r, b1r, a0, a1, f0, f1,
                     g0, g1, ss0, ss1, acc):
    cid = lax.axis_index("c")
    sid = lax.axis_index("s")

    _zero_acc(f0, acc, DW, sid)
    plsc.subcore_barrier()

    @pl.when(cid == 0)
    def _():
        _pipe_gat(sid, gsrc2, gdst2, xwt, adt,
                  ((s0, d0, b0r, a0, f0, g0, ss0),
                   (s1, d1, b1r, a1, f1, g1, ss1)),
                  acc)

    @pl.when(cid == 1)
    def _():
        tail = jnp.where(lax.iota(jnp.int32, 16) == 0, 1.0, 0.0)
        for fbuf in (f0, f1):
            for j in range(BG):
                fbuf[j, pl.ds(DI, 16)] = tail
        _pipe_plain(NBSA, sid, ssrc2, sdst2, xt,
                    ((s0, d0, b0r, f0, g0, ss0),
                     (s1, d1, b1r, f1, g1, ss1)), acc, DI)

    plsc.subcore_barrier()

    base = pl.multiple_of(sid * RPT, 16)

    @pl.when(cid == 0)
    def _():
        pltpu.sync_copy(acc.at[pl.ds(base, RPT)], md_out.at[pl.ds(base, RPT)])

    @pl.when(cid == 1)
    def _():
        pltpu.sync_copy(acc.at[pl.ds(base, RPT)], ag_out.at[pl.ds(base, RPT)])


def _sc_stage_a(xwt, xt, adt, gsrc2, gdst2, ssrc2, sdst2):
    mesh = plsc.VectorSubcoreMesh(core_axis_name="c", subcore_axis_name="s")
    return pl.kernel(
        _sc_stage_a_body,
        out_type=[
            jax.ShapeDtypeStruct((NP, DW), F32),
            jax.ShapeDtypeStruct((NP, DW), F32),
        ],
        mesh=mesh,
        compiler_params=pltpu.CompilerParams(
            needs_layout_passes=False, use_tc_tiling_on_sc=False),
        scratch_types=[
            pltpu.VMEM((BG,), jnp.int32),
            pltpu.VMEM((BG,), jnp.int32),
            pltpu.VMEM((BG,), jnp.int32),
            pltpu.VMEM((BG,), jnp.int32),
            pltpu.VMEM((BG, WI), jnp.int32),
            pltpu.VMEM((BG, WI), jnp.int32),
            pltpu.VMEM((BG, 16), F32),
            pltpu.VMEM((BG, 16), F32),
            pltpu.VMEM((BG, DW), F32),
            pltpu.VMEM((BG, DW), F32),
            pltpu.SemaphoreType.DMA,
            pltpu.SemaphoreType.DMA,
            pltpu.SemaphoreType.DMA,
            pltpu.SemaphoreType.DMA,
            pltpu.VMEM_SHARED((NP, DW), F32),
        ],
    )(xwt, xt, adt, gsrc2, gdst2, ssrc2, sdst2)


def _sc_stage_c_body(blo, bhi, ssrc2, sdst2, alo_out, ahi_out,
                     s0, s1, d0, d1, b0r, b1r, f0, f1,
                     g0, g1, ss0, ss1, acc):
    cid = lax.axis_index("c")
    sid = lax.axis_index("s")

    _zero_acc(f0, acc, H2, sid)
    plsc.subcore_barrier()

    bufs = ((s0, d0, b0r, f0, g0, ss0), (s1, d1, b1r, f1, g1, ss1))

    @pl.when(cid == 0)
    def _():
        _pipe_plain(NBC, sid, ssrc2, sdst2, blo, bufs, acc, H2)

    @pl.when(cid == 1)
    def _():
        _pipe_plain(NBC, sid, ssrc2, sdst2, bhi, bufs, acc, H2)

    plsc.subcore_barrier()

    base = pl.multiple_of(sid * RPT, 16)

    @pl.when(cid == 0)
    def _():
        pltpu.sync_copy(acc.at[pl.ds(base, RPT)], alo_out.at[pl.ds(base, RPT)])

    @pl.when(cid == 1)
    def _():
        pltpu.sync_copy(acc.at[pl.ds(base, RPT)], ahi_out.at[pl.ds(base, RPT)])


def _sc_stage_c(blo, bhi, ssrc2, sdst2):
    mesh = plsc.VectorSubcoreMesh(core_axis_name="c", subcore_axis_name="s")
    return pl.kernel(
        _sc_stage_c_body,
        out_type=[
            jax.ShapeDtypeStruct((NP, H2), F32),
            jax.ShapeDtypeStruct((NP, H2), F32),
        ],
        mesh=mesh,
        compiler_params=pltpu.CompilerParams(
            needs_layout_passes=False, use_tc_tiling_on_sc=False),
        scratch_types=[
            pltpu.VMEM((BC,), jnp.int32),
            pltpu.VMEM((BC,), jnp.int32),
            pltpu.VMEM((BC,), jnp.int32),
            pltpu.VMEM((BC,), jnp.int32),
            pltpu.VMEM((BC, H2 // 2), jnp.int32),
            pltpu.VMEM((BC, H2 // 2), jnp.int32),
            pltpu.VMEM((BC, H2), F32),
            pltpu.VMEM((BC, H2), F32),
            pltpu.SemaphoreType.DMA,
            pltpu.SemaphoreType.DMA,
            pltpu.SemaphoreType.DMA,
            pltpu.SemaphoreType.DMA,
            pltpu.VMEM_SHARED((NP, H2), F32),
        ],
    )(blo, bhi, ssrc2, sdst2)


def kernel(x, edge_index, W_gat, att_src, att_dst, b_gat, W1_l, b1_l, W1_r,
           g1, be1, W2_l, b2_l, W2_r, g2, be2, Wc, bc):
    src = edge_index[0]
    dst = edge_index[1]
    loops = jnp.arange(N, dtype=jnp.int32)

    gsrc = jnp.concatenate([src, loops])
    gdst = jnp.concatenate([dst, loops])
    gsrc2 = jnp.pad(gsrc, (0, EGP - EG)).reshape(NSUB * NBG, BG)
    gdst2 = jnp.pad(gdst, (0, EGP - EG), constant_values=N).reshape(NSUB * NBG, BG)
    ssrcA = jnp.pad(src, (0, ESPA - E)).reshape(NSUB * NBSA, BG)
    sdstA = jnp.pad(dst, (0, ESPA - E), constant_values=N).reshape(NSUB * NBSA, BG)
    ssrcC = jnp.pad(src, (0, ESPC - E)).reshape(NSUB * NBC, BC)
    sdstC = jnp.pad(dst, (0, ESPC - E), constant_values=N).reshape(NSUB * NBC, BC)

    xP = jnp.pad(x, ((0, NP - N), (0, 0)))

    xw, as2, ad2 = _tc_pre(xP, W_gat,
                           att_src.reshape(H2, 1), att_dst.reshape(H2, 1))

    zpad = jnp.zeros((NP, WI - DI // 2 - 1), jnp.int32)
    asw = _pack_words(jnp.concatenate([as2, jnp.zeros((NP, 1), F32)], axis=1))
    xwt = jnp.concatenate([_pack_words(xw), asw, zpad], axis=1)
    xt = jnp.concatenate(
        [_pack_words(xP), jnp.zeros((NP, WI - DI // 2), jnp.int32)], axis=1)
    adt = jnp.concatenate([ad2, jnp.zeros((NP, 15), F32)], axis=1)

    md, ag = _sc_stage_a(xwt, xt, adt, gsrc2, gdst2, ssrcA, sdstA)

    hlo, hhi = _tc_mid(
        md, ag, xP, b_gat[PERM].reshape(1, H2), W1_l[PERM, :],
        b1_l.reshape(1, H2), W1_r,
        jnp.concatenate([g1[:H2][PERM], g1[H2:]]).reshape(1, HID),
        jnp.concatenate([be1[:H2][PERM], be1[H2:]]).reshape(1, HID))

    W2r_p = jnp.concatenate([W2_r[:H2][PERM], W2_r[H2:]], axis=0)
    hr = _tc_hr(hlo, hhi, W2r_p, b2_l.reshape(1, HID))

    alo, ahi = _sc_stage_c(_pack_words(hlo), _pack_words(hhi), ssrcC, sdstC)

    W2l_p = jnp.concatenate([W2_l[:H2][PERM2], W2_l[H2:][PERM]], axis=0)
    deg = lax.slice(ag, (0, DI), (NP, DI + 1))
    out = _tc_fin(alo, ahi, deg, hr, W2l_p,
                  g2.reshape(1, HID), be2.reshape(1, HID), Wc,
                  bc.reshape(1, 1))
    return out[:N, 0]


# speedup vs baseline: 1.2101x; 1.2099x over previous
import functools

import numpy as _np

import jax
import jax.numpy as jnp
from jax import lax
from jax.experimental import pallas as pl
from jax.experimental.pallas import tpu as pltpu
from jax.experimental.pallas import tpu_sc as plsc

F32 = jnp.float32

N = 10000
E = 320000
DI = 128
H2 = 128
HID = 256

NSUB = 16
NCORE = 2
BG = 80
BC = 120
RPT = 640
NP = NSUB * RPT

DW = DI + 16
WI = DI // 2 + 8

_base = _np.arange(0, 32, 2)
_blk = _np.concatenate([_base, _base + 1])
PERM = _np.concatenate([_blk + 32 * k for k in range(4)])
PERM2 = PERM[PERM]


def _even_blocks(edges, be):
    nb = -(-edges // (NSUB * be))
    return nb + (nb % 2)


EG = E + N
NBG = _even_blocks(EG, BG)
NBSA = _even_blocks(E, BG)
EH = E // 2
NBC = -(-EH // (NSUB * BC))
NBC += NBC % 2
EGP = NSUB * NBG * BG
ESPA = NSUB * NBSA * BG
EHP = NSUB * NBC * BC

RB = 512
GRID = NP // RB


def _dot(a, b):
    return jnp.dot(a, b, preferred_element_type=F32)


def _tc_pre_body(x_ref, wg_ref, asrc_ref, adst_ref,
                 xw_ref, as_ref, ad_ref):
    x = x_ref[...]
    xw = _dot(x, wg_ref[...])
    xw_ref[...] = xw
    as_ref[...] = _dot(xw, asrc_ref[...])
    ad_ref[...] = _dot(xw, adst_ref[...])


def _tc_pre(xP, W_gat, a_src, a_dst):
    row = lambda i: (i, 0)
    full = lambda i: (0, 0)
    return pl.pallas_call(
        _tc_pre_body,
        grid=(GRID,),
        in_specs=[
            pl.BlockSpec((RB, DI), row),
            pl.BlockSpec((DI, H2), full),
            pl.BlockSpec((H2, 1), full),
            pl.BlockSpec((H2, 1), full),
        ],
        out_specs=[
            pl.BlockSpec((RB, DI), row),
            pl.BlockSpec((RB, 1), row),
            pl.BlockSpec((RB, 1), row),
        ],
        out_shape=[
            jax.ShapeDtypeStruct((NP, DI), F32),
            jax.ShapeDtypeStruct((NP, 1), F32),
            jax.ShapeDtypeStruct((NP, 1), F32),
        ],
    )(xP, W_gat, a_src, a_dst)


def _pack_words(a):
    ab = a.astype(jnp.bfloat16)
    return lax.bitcast_convert_type(
        ab.reshape(a.shape[0], a.shape[1] // 2, 2), jnp.int32)


def _layernorm(h, g, b):
    mu = jnp.mean(h, axis=1, keepdims=True)
    d = h - mu
    var = jnp.mean(d * d, axis=1, keepdims=True)
    return d * lax.rsqrt(var + 1e-5) * g + b


def _tc_mid_body(md_ref, ag_ref, x_ref, bgat_ref, w1l_ref, b1l_ref,
                 w1r_ref, g1_ref, be1_ref,
                 hlo_ref, hhi_ref, hb_ref):
    md = md_ref[...]
    ag = ag_ref[...]
    x1 = md[:, :H2] / jnp.maximum(md[:, H2:H2 + 1], 1e-16) + bgat_ref[...]
    mean1 = ag[:, :DI] / jnp.maximum(ag[:, DI:DI + 1], 1.0)
    x2 = _dot(mean1, w1l_ref[...]) + b1l_ref[...] + _dot(x_ref[...], w1r_ref[...])
    h = jnp.concatenate([x1, x2], axis=1)
    h = _layernorm(h, g1_ref[...], be1_ref[...])
    h = jnp.maximum(h, 0.0)
    hlo_ref[...] = h[:, :H2]
    hhi_ref[...] = h[:, H2:]
    hb_ref[...] = h.astype(jnp.bfloat16)


def _tc_mid(md, ag, xP, bgat, W1_l, b1l, W1_r, g1, be1):
    row = lambda i: (i, 0)
    full = lambda i: (0, 0)
    return pl.pallas_call(
        _tc_mid_body,
        grid=(GRID,),
        in_specs=[
            pl.BlockSpec((RB, DW), row),
            pl.BlockSpec((RB, DW), row),
            pl.BlockSpec((RB, DI), row),
            pl.BlockSpec((1, H2), full),
            pl.BlockSpec((DI, H2), full),
            pl.BlockSpec((1, H2), full),
            pl.BlockSpec((DI, H2), full),
            pl.BlockSpec((1, HID), full),
            pl.BlockSpec((1, HID), full),
        ],
        out_specs=[
            pl.BlockSpec((RB, H2), row),
            pl.BlockSpec((RB, H2), row),
            pl.BlockSpec((RB, HID), row),
        ],
        out_shape=[
            jax.ShapeDtypeStruct((NP, H2), F32),
            jax.ShapeDtypeStruct((NP, H2), F32),
            jax.ShapeDtypeStruct((NP, HID), jnp.bfloat16),
        ],
    )(md, ag, xP, bgat, W1_l, b1l, W1_r, g1, be1)


def _tc_hr_body(hlo_ref, hhi_ref, w2r_ref, b2l_ref, hr_ref):
    w2r = w2r_ref[...]
    hr_ref[...] = (_dot(hlo_ref[...], w2r[:H2, :])
                   + _dot(hhi_ref[...], w2r[H2:, :]) + b2l_ref[...])


def _tc_hr(hlo, hhi, W2_r, b2l):
    row = lambda i: (i, 0)
    full = lambda i: (0, 0)
    return pl.pallas_call(
        _tc_hr_body,
        grid=(GRID,),
        in_specs=[
            pl.BlockSpec((RB, H2), row),
            pl.BlockSpec((RB, H2), row),
            pl.BlockSpec((HID, HID), full),
            pl.BlockSpec((1, HID), full),
        ],
        out_specs=pl.BlockSpec((RB, HID), row),
        out_shape=jax.ShapeDtypeStruct((NP, HID), F32),
    )(hlo, hhi, W2_r, b2l)


def _tc_fin_body(a0_ref, a1_ref, deg_ref, hr_ref, w2l_ref,
                 g2_ref, be2_ref, wc_ref, bc_ref, out_ref):
    deg = jnp.maximum(deg_ref[...], 1.0)
    agg2 = a0_ref[...].astype(F32) + a1_ref[...].astype(F32)
    h2 = _dot(agg2 / deg, w2l_ref[...]) + hr_ref[...]
    h2 = _layernorm(h2, g2_ref[...], be2_ref[...])
    h2 = jnp.maximum(h2, 0.0)
    out_ref[...] = _dot(h2, wc_ref[...]) + bc_ref[...]


def _tc_fin(a0, a1, deg, hr, W2_l, g2, be2, Wc, bc):
    row = lambda i: (i, 0)
    full = lambda i: (0, 0)
    return pl.pallas_call(
        _tc_fin_body,
        grid=(GRID,),
        in_specs=[
            pl.BlockSpec((RB, HID), row),
            pl.BlockSpec((RB, HID), row),
            pl.BlockSpec((RB, 1), row),
            pl.BlockSpec((RB, HID), row),
            pl.BlockSpec((HID, HID), full),
            pl.BlockSpec((1, HID), full),
            pl.BlockSpec((1, HID), full),
            pl.BlockSpec((HID, 1), full),
            pl.BlockSpec((1, 1), full),
        ],
        out_specs=pl.BlockSpec((RB, 1), row),
        out_shape=jax.ShapeDtypeStruct((NP, 1), F32),
    )(a0, a1, deg, hr, W2_l, g2, be2, Wc, bc)


def _zero_acc(r0, acc, width, sid):
    zv = jnp.zeros((16,), F32)
    for r in range(16):
        for k in range(width // 16):
            r0[r, pl.ds(k * 16, 16)] = zv
    base = sid * RPT

    def body(i, _):
        off = pl.multiple_of(base + i * 16, 16)
        pltpu.sync_copy(r0.at[pl.ds(0, 16)], acc.at[pl.ds(off, 16)])
        return 0

    lax.fori_loop(0, RPT // 16, body, 0)


def _unpack32(v):
    fa = plsc.bitcast(v << 16, F32)
    fb = plsc.bitcast(v & jnp.int32(-65536), F32)
    return fa, fb


def _pipe_plain(nblk, sid, src2, dst2, table, bufs, acc, width):
    (s0, d0, b0r, f0, g0, ss0), (s1, d1, b1r, f1, g1, ss1) = bufs
    half = nblk // 2
    nb = src2.shape[1]
    wcols = width // 32

    def fire(bid, sidx, didx, gsem, braw):
        pltpu.sync_copy(src2.at[bid], sidx)
        pltpu.sync_copy(dst2.at[bid], didx)
        pltpu.async_copy(table.at[sidx], braw, gsem)

    def process(sidx, didx, gsem, braw, fbuf, ssem):
        pltpu.make_async_copy(table.at[sidx], braw, gsem).wait()
        for j in range(nb):
            for k in range(wcols):
                fa, fb = _unpack32(braw[j, pl.ds(k * 16, 16)])
                fbuf[j, pl.ds(k * 32, 16)] = fa
                fbuf[j, pl.ds(k * 32 + 16, 16)] = fb
        pltpu.async_copy(fbuf, acc.at[didx], ssem, add=True)

    fire(sid * nblk, s0, d0, g0, b0r)
    fire(sid * nblk + 1, s1, d1, g1, b1r)

    def body(i, _):
        bb = sid * nblk + 2 * i
        process(s0, d0, g0, b0r, f0, ss0)
        process(s1, d1, g1, b1r, f1, ss1)

        @pl.when(i < half - 1)
        def _():
            pltpu.make_async_copy(f0, acc.at[d0], ss0).wait()
            fire(bb + 2, s0, d0, g0, b0r)
            pltpu.make_async_copy(f1, acc.at[d1], ss1).wait()
            fire(bb + 3, s1, d1, g1, b1r)

        return 0

    lax.fori_loop(0, half, body, 0)
    pltpu.make_async_copy(f0, acc.at[d0], ss0).wait()
    pltpu.make_async_copy(f1, acc.at[d1], ss1).wait()


def _pipe_gat(sid, src2, dst2, xwt, adt, bufs, acc):
    (s0, d0, b0r, a0, f0, g0, ss0), (s1, d1, b1r, a1, f1, g1, ss1) = bufs
    half = NBG // 2
    lane = lax.iota(jnp.int32, 16)
    as_col = jnp.full((16,), DI // 2, jnp.int32)
    zero_col = jnp.zeros((16,), jnp.int32)

    def fire(bid, sidx, didx, gsem, braw, adr):
        pltpu.sync_copy(src2.at[bid], sidx)
        pltpu.sync_copy(dst2.at[bid], didx)
        pltpu.async_copy(xwt.at[sidx], braw, gsem)
        pltpu.async_copy(adt.at[didx], adr, gsem)

    def process(sidx, didx, gsem, braw, adr, fbuf, ssem):
        pltpu.make_async_copy(xwt.at[sidx], braw, gsem).wait()
        pltpu.make_async_copy(adt.at[didx], adr, gsem).wait()
        for j8 in range(BG // 16):
            jvec = lane + (j8 * 16)
            as_f, _ = _unpack32(plsc.load_gather(braw, [jvec, as_col]))
            ad_f = plsc.load_gather(adr, [jvec, zero_col])
            u = as_f + ad_f
            wv = jnp.exp(jnp.maximum(u, 0.2 * u))
            for l in range(16):
                w = wv[l]
                j = j8 * 16 + l
                for k in range(DI // 32):
                    fa, fb = _unpack32(braw[j, pl.ds(k * 16, 16)])
                    fbuf[j, pl.ds(k * 32, 16)] = fa * w
                    fbuf[j, pl.ds(k * 32 + 16, 16)] = fb * w
                fbuf[j, pl.ds(DI, 16)] = lax.broadcast(w, (16,))
        pltpu.async_copy(fbuf, acc.at[didx], ssem, add=True)

    fire(sid * NBG, s0, d0, g0, b0r, a0)
    fire(sid * NBG + 1, s1, d1, g1, b1r, a1)

    def body(i, _):
        bb = sid * NBG + 2 * i
        process(s0, d0, g0, b0r, a0, f0, ss0)
        process(s1, d1, g1, b1r, a1, f1, ss1)

        @pl.when(i < half - 1)
        def _():
            pltpu.make_async_copy(f0, acc.at[d0], ss0).wait()
            fire(bb + 2, s0, d0, g0, b0r, a0)
            pltpu.make_async_copy(f1, acc.at[d1], ss1).wait()
            fire(bb + 3, s1, d1, g1, b1r, a1)

        return 0

    lax.fori_loop(0, half, body, 0)
    pltpu.make_async_copy(f0, acc.at[d0], ss0).wait()
    pltpu.make_async_copy(f1, acc.at[d1], ss1).wait()


def _sc_stage_a_body(xwt, xt, adt, gsrc2, gdst2, ssrc2, sdst2,
                     md_out, ag_out,
                     s0, s1, d0, d1, b0r, b1r, a0, a1, f0, f1,
                     g0, g1, ss0, ss1, acc):
    cid = lax.axis_index("c")
    sid = lax.axis_index("s")

    _zero_acc(f0, acc, DW, sid)
    plsc.subcore_barrier()

    @pl.when(cid == 0)
    def _():
        _pipe_gat(sid, gsrc2, gdst2, xwt, adt,
                  ((s0, d0, b0r, a0, f0, g0, ss0),
                   (s1, d1, b1r, a1, f1, g1, ss1)),
                  acc)

    @pl.when(cid == 1)
    def _():
        tail = jnp.where(lax.iota(jnp.int32, 16) == 0, 1.0, 0.0)
        for fbuf in (f0, f1):
            for j in range(BG):
                fbuf[j, pl.ds(DI, 16)] = tail
        _pipe_plain(NBSA, sid, ssrc2, sdst2, xt,
                    ((s0, d0, b0r, f0, g0, ss0),
                     (s1, d1, b1r, f1, g1, ss1)), acc, DI)

    plsc.subcore_barrier()

    base = pl.multiple_of(sid * RPT, 16)

    @pl.when(cid == 0)
    def _():
        pltpu.sync_copy(acc.at[pl.ds(base, RPT)], md_out.at[pl.ds(base, RPT)])

    @pl.when(cid == 1)
    def _():
        pltpu.sync_copy(acc.at[pl.ds(base, RPT)], ag_out.at[pl.ds(base, RPT)])


def _sc_stage_a(xwt, xt, adt, gsrc2, gdst2, ssrc2, sdst2):
    mesh = plsc.VectorSubcoreMesh(core_axis_name="c", subcore_axis_name="s")
    return pl.kernel(
        _sc_stage_a_body,
        out_type=[
            jax.ShapeDtypeStruct((NP, DW), F32),
            jax.ShapeDtypeStruct((NP, DW), F32),
        ],
        mesh=mesh,
        compiler_params=pltpu.CompilerParams(
            needs_layout_passes=False, use_tc_tiling_on_sc=False),
        scratch_types=[
            pltpu.VMEM((BG,), jnp.int32),
            pltpu.VMEM((BG,), jnp.int32),
            pltpu.VMEM((BG,), jnp.int32),
            pltpu.VMEM((BG,), jnp.int32),
            pltpu.VMEM((BG, WI), jnp.int32),
            pltpu.VMEM((BG, WI), jnp.int32),
            pltpu.VMEM((BG, 16), F32),
            pltpu.VMEM((BG, 16), F32),
            pltpu.VMEM((BG, DW), F32),
            pltpu.VMEM((BG, DW), F32),
            pltpu.SemaphoreType.DMA,
            pltpu.SemaphoreType.DMA,
            pltpu.SemaphoreType.DMA,
            pltpu.SemaphoreType.DMA,
            pltpu.VMEM_SHARED((NP, DW), F32),
        ],
    )(xwt, xt, adt, gsrc2, gdst2, ssrc2, sdst2)


def _sc_stage_c_body(ht, srcC, dstC, a_out0, a_out1,
                     s0, s1, d0, d1, b0r, b1r, g0, g1, ss0, ss1, accb):
    cid = lax.axis_index("c")
    sid = lax.axis_index("s")

    zv = jnp.zeros((32,), jnp.bfloat16)
    for r in range(16):
        for k in range(HID // 32):
            b0r[r, pl.ds(k * 32, 32)] = zv
    base = pl.multiple_of(sid * RPT, 16)

    def zbody(i, _):
        off = pl.multiple_of(base + i * 16, 16)
        pltpu.sync_copy(b0r.at[pl.ds(0, 16)], accb.at[pl.ds(off, 16)])
        return 0

    lax.fori_loop(0, RPT // 16, zbody, 0)
    plsc.subcore_barrier()

    tbase = (cid * NSUB + sid) * NBC
    half = NBC // 2

    def fire(bid, sidx, didx, gsem, braw):
        pltpu.sync_copy(srcC.at[bid], sidx)
        pltpu.sync_copy(dstC.at[bid], didx)
        pltpu.async_copy(ht.at[sidx], braw, gsem)

    fire(tbase, s0, d0, g0, b0r)
    fire(tbase + 1, s1, d1, g1, b1r)

    def body(i, _):
        bb = tbase + 2 * i
        pltpu.make_async_copy(ht.at[s0], b0r, g0).wait()
        pltpu.async_copy(b0r, accb.at[d0], ss0, add=True)
        pltpu.make_async_copy(ht.at[s1], b1r, g1).wait()
        pltpu.async_copy(b1r, accb.at[d1], ss1, add=True)

        @pl.when(i < half - 1)
        def _():
            pltpu.make_async_copy(b0r, accb.at[d0], ss0).wait()
            fire(bb + 2, s0, d0, g0, b0r)
            pltpu.make_async_copy(b1r, accb.at[d1], ss1).wait()
            fire(bb + 3, s1, d1, g1, b1r)

        return 0

    lax.fori_loop(0, half, body, 0)
    pltpu.make_async_copy(b0r, accb.at[d0], ss0).wait()
    pltpu.make_async_copy(b1r, accb.at[d1], ss1).wait()

    plsc.subcore_barrier()

    @pl.when(cid == 0)
    def _():
        pltpu.sync_copy(accb.at[pl.ds(base, RPT)], a_out0.at[pl.ds(base, RPT)])

    @pl.when(cid == 1)
    def _():
        pltpu.sync_copy(accb.at[pl.ds(base, RPT)], a_out1.at[pl.ds(base, RPT)])


def _sc_stage_c(ht, srcC, dstC):
    mesh = plsc.VectorSubcoreMesh(core_axis_name="c", subcore_axis_name="s")
    return pl.kernel(
        _sc_stage_c_body,
        out_type=[
            jax.ShapeDtypeStruct((NP, HID), jnp.bfloat16),
            jax.ShapeDtypeStruct((NP, HID), jnp.bfloat16),
        ],
        mesh=mesh,
        compiler_params=pltpu.CompilerParams(
            needs_layout_passes=False, use_tc_tiling_on_sc=False),
        scratch_types=[
            pltpu.VMEM((BC,), jnp.int32),
            pltpu.VMEM((BC,), jnp.int32),
            pltpu.VMEM((BC,), jnp.int32),
            pltpu.VMEM((BC,), jnp.int32),
            pltpu.VMEM((BC, HID), jnp.bfloat16),
            pltpu.VMEM((BC, HID), jnp.bfloat16),
            pltpu.SemaphoreType.DMA,
            pltpu.SemaphoreType.DMA,
            pltpu.SemaphoreType.DMA,
            pltpu.SemaphoreType.DMA,
            pltpu.VMEM_SHARED((NP, HID), jnp.bfloat16),
        ],
    )(ht, srcC, dstC)


def kernel(x, edge_index, W_gat, att_src, att_dst, b_gat, W1_l, b1_l, W1_r,
           g1, be1, W2_l, b2_l, W2_r, g2, be2, Wc, bc):
    src = edge_index[0]
    dst = edge_index[1]
    loops = jnp.arange(N, dtype=jnp.int32)

    gsrc = jnp.concatenate([src, loops])
    gdst = jnp.concatenate([dst, loops])
    gsrc2 = jnp.pad(gsrc, (0, EGP - EG)).reshape(NSUB * NBG, BG)
    gdst2 = jnp.pad(gdst, (0, EGP - EG), constant_values=N).reshape(NSUB * NBG, BG)
    ssrcA = jnp.pad(src, (0, ESPA - E)).reshape(NSUB * NBSA, BG)
    sdstA = jnp.pad(dst, (0, ESPA - E), constant_values=N).reshape(NSUB * NBSA, BG)
    srcC = jnp.concatenate([
        jnp.pad(src[:EH], (0, EHP - EH)),
        jnp.pad(src[EH:], (0, EHP - EH))]).reshape(2 * NSUB * NBC, BC)
    dstC = jnp.concatenate([
        jnp.pad(dst[:EH], (0, EHP - EH), constant_values=N),
        jnp.pad(dst[EH:], (0, EHP - EH), constant_values=N),
    ]).reshape(2 * NSUB * NBC, BC)

    xP = jnp.pad(x, ((0, NP - N), (0, 0)))

    xw, as2, ad2 = _tc_pre(xP, W_gat,
                           att_src.reshape(H2, 1), att_dst.reshape(H2, 1))

    zpad = jnp.zeros((NP, WI - DI // 2 - 1), jnp.int32)
    asw = _pack_words(jnp.concatenate([as2, jnp.zeros((NP, 1), F32)], axis=1))
    xwt = jnp.concatenate([_pack_words(xw), asw, zpad], axis=1)
    xt = jnp.concatenate(
        [_pack_words(xP), jnp.zeros((NP, WI - DI // 2), jnp.int32)], axis=1)
    adt = jnp.concatenate([ad2, jnp.zeros((NP, 15), F32)], axis=1)

    md, ag = _sc_stage_a(xwt, xt, adt, gsrc2, gdst2, ssrcA, sdstA)

    hlo, hhi, hb = _tc_mid(
        md, ag, xP, b_gat[PERM].reshape(1, H2), W1_l[PERM, :],
        b1_l.reshape(1, H2), W1_r,
        jnp.concatenate([g1[:H2][PERM], g1[H2:]]).reshape(1, HID),
        jnp.concatenate([be1[:H2][PERM], be1[H2:]]).reshape(1, HID))

    W2r_p = jnp.concatenate([W2_r[:H2][PERM], W2_r[H2:]], axis=0)
    hr = _tc_hr(hlo, hhi, W2r_p, b2_l.reshape(1, HID))

    a0, a1 = _sc_stage_c(hb, srcC, dstC)

    W2l_p = jnp.concatenate([W2_l[:H2][PERM], W2_l[H2:]], axis=0)
    deg = lax.slice(ag, (0, DI), (NP, DI + 1))
    out = _tc_fin(a0, a1, deg, hr, W2l_p,
                  g2.reshape(1, HID), be2.reshape(1, HID), Wc,
                  bc.reshape(1, 1))
    return out[:N, 0]


# speedup vs baseline: 1.2612x; 1.0422x over previous
import functools

import numpy as _np

import jax
import jax.numpy as jnp
from jax import lax
from jax.experimental import pallas as pl
from jax.experimental.pallas import tpu as pltpu
from jax.experimental.pallas import tpu_sc as plsc

F32 = jnp.float32

N = 10000
E = 320000
DI = 128
H2 = 128
HID = 256

NSUB = 16
NCORE = 2
BG = 64
BC = 120
RPT = 640
NP = NSUB * RPT

DW = DI + 16
WI = DI // 2 + 8

_base = _np.arange(0, 32, 2)
_blk = _np.concatenate([_base, _base + 1])
PERM = _np.concatenate([_blk + 32 * k for k in range(4)])
PERM2 = PERM[PERM]


def _even_blocks(edges, be):
    nb = -(-edges // (NSUB * be))
    return nb + (nb % 2)


EG = E + N
NBG = _even_blocks(EG, BG)
NBSA = _even_blocks(E, BG)
EH = E // 2
NBC = -(-EH // (NSUB * BC))
NBC += NBC % 2
EGP = NSUB * NBG * BG
ESPA = NSUB * NBSA * BG
EHP = NSUB * NBC * BC

RB = 512
GRID = NP // RB


def _dot(a, b):
    return jnp.dot(a, b, preferred_element_type=F32)


def _tc_pre_body(x_ref, wg_ref, asrc_ref, adst_ref,
                 xw_ref, as_ref, ad_ref):
    x = x_ref[...]
    xw = _dot(x, wg_ref[...])
    xw_ref[...] = xw
    as_ref[...] = _dot(xw, asrc_ref[...])
    ad_ref[...] = _dot(xw, adst_ref[...])


def _tc_pre(xP, W_gat, a_src, a_dst):
    row = lambda i: (i, 0)
    full = lambda i: (0, 0)
    return pl.pallas_call(
        _tc_pre_body,
        grid=(GRID,),
        in_specs=[
            pl.BlockSpec((RB, DI), row),
            pl.BlockSpec((DI, H2), full),
            pl.BlockSpec((H2, 1), full),
            pl.BlockSpec((H2, 1), full),
        ],
        out_specs=[
            pl.BlockSpec((RB, DI), row),
            pl.BlockSpec((RB, 1), row),
            pl.BlockSpec((RB, 1), row),
        ],
        out_shape=[
            jax.ShapeDtypeStruct((NP, DI), F32),
            jax.ShapeDtypeStruct((NP, 1), F32),
            jax.ShapeDtypeStruct((NP, 1), F32),
        ],
    )(xP, W_gat, a_src, a_dst)


def _pack_words(a):
    ab = a.astype(jnp.bfloat16)
    return lax.bitcast_convert_type(
        ab.reshape(a.shape[0], a.shape[1] // 2, 2), jnp.int32)


def _layernorm(h, g, b):
    mu = jnp.mean(h, axis=1, keepdims=True)
    d = h - mu
    var = jnp.mean(d * d, axis=1, keepdims=True)
    return d * lax.rsqrt(var + 1e-5) * g + b


def _tc_mid_body(md_ref, ag_ref, x_ref, bgat_ref, w1l_ref, b1l_ref,
                 w1r_ref, g1_ref, be1_ref,
                 hlo_ref, hhi_ref, hb_ref):
    md = md_ref[...]
    ag = ag_ref[...]
    x1 = md[:, :H2] / jnp.maximum(md[:, H2:H2 + 1], 1e-16) + bgat_ref[...]
    mean1 = ag[:, :DI] / jnp.maximum(ag[:, DI:DI + 1], 1.0)
    x2 = _dot(mean1, w1l_ref[...]) + b1l_ref[...] + _dot(x_ref[...], w1r_ref[...])
    h = jnp.concatenate([x1, x2], axis=1)
    h = _layernorm(h, g1_ref[...], be1_ref[...])
    h = jnp.maximum(h, 0.0)
    hlo_ref[...] = h[:, :H2]
    hhi_ref[...] = h[:, H2:]
    hb_ref[...] = h.astype(jnp.bfloat16)


def _tc_mid(md, ag, xP, bgat, W1_l, b1l, W1_r, g1, be1):
    row = lambda i: (i, 0)
    full = lambda i: (0, 0)
    return pl.pallas_call(
        _tc_mid_body,
        grid=(GRID,),
        in_specs=[
            pl.BlockSpec((RB, DW), row),
            pl.BlockSpec((RB, DW), row),
            pl.BlockSpec((RB, DI), row),
            pl.BlockSpec((1, H2), full),
            pl.BlockSpec((DI, H2), full),
            pl.BlockSpec((1, H2), full),
            pl.BlockSpec((DI, H2), full),
            pl.BlockSpec((1, HID), full),
            pl.BlockSpec((1, HID), full),
        ],
        out_specs=[
            pl.BlockSpec((RB, H2), row),
            pl.BlockSpec((RB, H2), row),
            pl.BlockSpec((RB, HID), row),
        ],
        out_shape=[
            jax.ShapeDtypeStruct((NP, H2), F32),
            jax.ShapeDtypeStruct((NP, H2), F32),
            jax.ShapeDtypeStruct((NP, HID), jnp.bfloat16),
        ],
    )(md, ag, xP, bgat, W1_l, b1l, W1_r, g1, be1)


def _tc_hr_body(hlo_ref, hhi_ref, w2r_ref, b2l_ref, hr_ref):
    w2r = w2r_ref[...]
    hr_ref[...] = (_dot(hlo_ref[...], w2r[:H2, :])
                   + _dot(hhi_ref[...], w2r[H2:, :]) + b2l_ref[...])


def _tc_hr(hlo, hhi, W2_r, b2l):
    row = lambda i: (i, 0)
    full = lambda i: (0, 0)
    return pl.pallas_call(
        _tc_hr_body,
        grid=(GRID,),
        in_specs=[
            pl.BlockSpec((RB, H2), row),
            pl.BlockSpec((RB, H2), row),
            pl.BlockSpec((HID, HID), full),
            pl.BlockSpec((1, HID), full),
        ],
        out_specs=pl.BlockSpec((RB, HID), row),
        out_shape=jax.ShapeDtypeStruct((NP, HID), F32),
    )(hlo, hhi, W2_r, b2l)


def _tc_fin_body(a0_ref, a1_ref, deg_ref, hr_ref, w2l_ref,
                 g2_ref, be2_ref, wc_ref, bc_ref, out_ref):
    deg = jnp.maximum(deg_ref[...], 1.0)
    agg2 = a0_ref[...].astype(F32) + a1_ref[...].astype(F32)
    h2 = _dot(agg2 / deg, w2l_ref[...]) + hr_ref[...]
    h2 = _layernorm(h2, g2_ref[...], be2_ref[...])
    h2 = jnp.maximum(h2, 0.0)
    out_ref[...] = _dot(h2, wc_ref[...]) + bc_ref[...]


def _tc_fin(a0, a1, deg, hr, W2_l, g2, be2, Wc, bc):
    row = lambda i: (i, 0)
    full = lambda i: (0, 0)
    return pl.pallas_call(
        _tc_fin_body,
        grid=(GRID,),
        in_specs=[
            pl.BlockSpec((RB, HID), row),
            pl.BlockSpec((RB, HID), row),
            pl.BlockSpec((RB, 1), row),
            pl.BlockSpec((RB, HID), row),
            pl.BlockSpec((HID, HID), full),
            pl.BlockSpec((1, HID), full),
            pl.BlockSpec((1, HID), full),
            pl.BlockSpec((HID, 1), full),
            pl.BlockSpec((1, 1), full),
        ],
        out_specs=pl.BlockSpec((RB, 1), row),
        out_shape=jax.ShapeDtypeStruct((NP, 1), F32),
    )(a0, a1, deg, hr, W2_l, g2, be2, Wc, bc)


def _zero_acc(r0, acc, width, sid):
    zv = jnp.zeros((16,), F32)
    for r in range(16):
        for k in range(width // 16):
            r0[r, pl.ds(k * 16, 16)] = zv
    base = sid * RPT

    def body(i, _):
        off = pl.multiple_of(base + i * 16, 16)
        pltpu.sync_copy(r0.at[pl.ds(0, 16)], acc.at[pl.ds(off, 16)])
        return 0

    lax.fori_loop(0, RPT // 16, body, 0)


def _unpack32(v):
    fa = plsc.bitcast(v << 16, F32)
    fb = plsc.bitcast(v & jnp.int32(-65536), F32)
    return fa, fb


def _pipe_plain(nblk, sid, src2, dst2, table, bufs, acc, width):
    (s0, d0, b0r, f0, g0, ss0), (s1, d1, b1r, f1, g1, ss1) = bufs
    half = nblk // 2
    nb = src2.shape[1]
    wcols = width // 32

    def fire(bid, sidx, didx, gsem, braw):
        pltpu.sync_copy(src2.at[bid], sidx)
        pltpu.sync_copy(dst2.at[bid], didx)
        pltpu.async_copy(table.at[sidx], braw, gsem)

    def process(sidx, didx, gsem, braw, fbuf, ssem):
        pltpu.make_async_copy(table.at[sidx], braw, gsem).wait()
        for j in range(nb):
            for k in range(wcols):
                fa, fb = _unpack32(braw[j, pl.ds(k * 16, 16)])
                fbuf[j, pl.ds(k * 32, 16)] = fa
                fbuf[j, pl.ds(k * 32 + 16, 16)] = fb
        pltpu.async_copy(fbuf, acc.at[didx], ssem, add=True)

    fire(sid * nblk, s0, d0, g0, b0r)
    fire(sid * nblk + 1, s1, d1, g1, b1r)

    def body(i, _):
        bb = sid * nblk + 2 * i
        process(s0, d0, g0, b0r, f0, ss0)
        process(s1, d1, g1, b1r, f1, ss1)

        @pl.when(i < half - 1)
        def _():
            pltpu.make_async_copy(f0, acc.at[d0], ss0).wait()
            fire(bb + 2, s0, d0, g0, b0r)
            pltpu.make_async_copy(f1, acc.at[d1], ss1).wait()
            fire(bb + 3, s1, d1, g1, b1r)

        return 0

    lax.fori_loop(0, half, body, 0)
    pltpu.make_async_copy(f0, acc.at[d0], ss0).wait()
    pltpu.make_async_copy(f1, acc.at[d1], ss1).wait()


def _pipe_gat(sid, src2, dst2, xwt, adb, bufs, acc):
    (s0, d0, b0r, f0, g0, ss0), (s1, d1, b1r, f1, g1, ss1) = bufs
    half = NBG // 2
    lane = lax.iota(jnp.int32, 16)
    as_col = jnp.full((16,), DI // 2, jnp.int32)

    def fire(bid, sidx, didx, gsem, braw):
        pltpu.sync_copy(src2.at[bid], sidx)
        pltpu.sync_copy(dst2.at[bid], didx)
        pltpu.async_copy(xwt.at[sidx], braw, gsem)

    def process(sidx, didx, gsem, braw, fbuf, ssem):
        pltpu.make_async_copy(xwt.at[sidx], braw, gsem).wait()
        for j8 in range(BG // 16):
            jvec = lane + (j8 * 16)
            as_f, _ = _unpack32(plsc.load_gather(braw, [jvec, as_col]))
            ad_f = plsc.load_gather(adb, [didx[pl.ds(j8 * 16, 16)]])
            u = as_f + ad_f
            wv = jnp.exp(jnp.maximum(u, 0.2 * u))
            for l in range(16):
                w = wv[l]
                j = j8 * 16 + l
                for k in range(DI // 32):
                    fa, fb = _unpack32(braw[j, pl.ds(k * 16, 16)])
                    fbuf[j, pl.ds(k * 32, 16)] = fa * w
                    fbuf[j, pl.ds(k * 32 + 16, 16)] = fb * w
                fbuf[j, pl.ds(DI, 16)] = lax.broadcast(w, (16,))
        pltpu.async_copy(fbuf, acc.at[didx], ssem, add=True)

    fire(sid * NBG, s0, d0, g0, b0r)
    fire(sid * NBG + 1, s1, d1, g1, b1r)

    def body(i, _):
        bb = sid * NBG + 2 * i
        process(s0, d0, g0, b0r, f0, ss0)
        process(s1, d1, g1, b1r, f1, ss1)

        @pl.when(i < half - 1)
        def _():
            pltpu.make_async_copy(f0, acc.at[d0], ss0).wait()
            fire(bb + 2, s0, d0, g0, b0r)
            pltpu.make_async_copy(f1, acc.at[d1], ss1).wait()
            fire(bb + 3, s1, d1, g1, b1r)

        return 0

    lax.fori_loop(0, half, body, 0)
    pltpu.make_async_copy(f0, acc.at[d0], ss0).wait()
    pltpu.make_async_copy(f1, acc.at[d1], ss1).wait()


def _sc_stage_a_body(xwt, xt, ad1, gsrc2, gdst2, ssrc2, sdst2,
                     md_out, ag_out,
                     s0, s1, d0, d1, b0r, b1r, adb, f0, f1,
                     g0, g1, ss0, ss1, acc):
    cid = lax.axis_index("c")
    sid = lax.axis_index("s")

    _zero_acc(f0, acc, DW, sid)
    plsc.subcore_barrier()

    @pl.when(cid == 0)
    def _():
        pltpu.sync_copy(ad1, adb)
        _pipe_gat(sid, gsrc2, gdst2, xwt, adb,
                  ((s0, d0, b0r, f0, g0, ss0),
                   (s1, d1, b1r, f1, g1, ss1)),
                  acc)

    @pl.when(cid == 1)
    def _():
        tail = jnp.where(lax.iota(jnp.int32, 16) == 0, 1.0, 0.0)
        for fbuf in (f0, f1):
            for j in range(BG):
                fbuf[j, pl.ds(DI, 16)] = tail
        _pipe_plain(NBSA, sid, ssrc2, sdst2, xt,
                    ((s0, d0, b0r, f0, g0, ss0),
                     (s1, d1, b1r, f1, g1, ss1)), acc, DI)

    plsc.subcore_barrier()

    base = pl.multiple_of(sid * RPT, 16)

    @pl.when(cid == 0)
    def _():
        pltpu.sync_copy(acc.at[pl.ds(base, RPT)], md_out.at[pl.ds(base, RPT)])

    @pl.when(cid == 1)
    def _():
        pltpu.sync_copy(acc.at[pl.ds(base, RPT)], ag_out.at[pl.ds(base, RPT)])


def _sc_stage_a(xwt, xt, ad1, gsrc2, gdst2, ssrc2, sdst2):
    mesh = plsc.VectorSubcoreMesh(core_axis_name="c", subcore_axis_name="s")
    return pl.kernel(
        _sc_stage_a_body,
        out_type=[
            jax.ShapeDtypeStruct((NP, DW), F32),
            jax.ShapeDtypeStruct((NP, DW), F32),
        ],
        mesh=mesh,
        compiler_params=pltpu.CompilerParams(
            needs_layout_passes=False, use_tc_tiling_on_sc=False),
        scratch_types=[
            pltpu.VMEM((BG,), jnp.int32),
            pltpu.VMEM((BG,), jnp.int32),
            pltpu.VMEM((BG,), jnp.int32),
            pltpu.VMEM((BG,), jnp.int32),
            pltpu.VMEM((BG, WI), jnp.int32),
            pltpu.VMEM((BG, WI), jnp.int32),
            pltpu.VMEM((NP,), F32),
            pltpu.VMEM((BG, DW), F32),
            pltpu.VMEM((BG, DW), F32),
            pltpu.SemaphoreType.DMA,
            pltpu.SemaphoreType.DMA,
            pltpu.SemaphoreType.DMA,
            pltpu.SemaphoreType.DMA,
            pltpu.VMEM_SHARED((NP, DW), F32),
        ],
    )(xwt, xt, ad1, gsrc2, gdst2, ssrc2, sdst2)


def _sc_stage_c_body(ht, srcC, dstC, a_out0, a_out1,
                     s0, s1, d0, d1, b0r, b1r, g0, g1, ss0, ss1, accb):
    cid = lax.axis_index("c")
    sid = lax.axis_index("s")

    zv = jnp.zeros((32,), jnp.bfloat16)
    for r in range(16):
        for k in range(HID // 32):
            b0r[r, pl.ds(k * 32, 32)] = zv
    base = pl.multiple_of(sid * RPT, 16)

    def zbody(i, _):
        off = pl.multiple_of(base + i * 16, 16)
        pltpu.sync_copy(b0r.at[pl.ds(0, 16)], accb.at[pl.ds(off, 16)])
        return 0

    lax.fori_loop(0, RPT // 16, zbody, 0)
    plsc.subcore_barrier()

    tbase = (cid * NSUB + sid) * NBC
    half = NBC // 2

    def fire(bid, sidx, didx, gsem, braw):
        pltpu.sync_copy(srcC.at[bid], sidx)
        pltpu.sync_copy(dstC.at[bid], didx)
        pltpu.async_copy(ht.at[sidx], braw, gsem)

    fire(tbase, s0, d0, g0, b0r)
    fire(tbase + 1, s1, d1, g1, b1r)

    def body(i, _):
        bb = tbase + 2 * i
        pltpu.make_async_copy(ht.at[s0], b0r, g0).wait()
        pltpu.async_copy(b0r, accb.at[d0], ss0, add=True)
        pltpu.make_async_copy(ht.at[s1], b1r, g1).wait()
        pltpu.async_copy(b1r, accb.at[d1], ss1, add=True)

        @pl.when(i < half - 1)
        def _():
            pltpu.make_async_copy(b0r, accb.at[d0], ss0).wait()
            fire(bb + 2, s0, d0, g0, b0r)
            pltpu.make_async_copy(b1r, accb.at[d1], ss1).wait()
            fire(bb + 3, s1, d1, g1, b1r)

        return 0

    lax.fori_loop(0, half, body, 0)
    pltpu.make_async_copy(b0r, accb.at[d0], ss0).wait()
    pltpu.make_async_copy(b1r, accb.at[d1], ss1).wait()

    plsc.subcore_barrier()

    @pl.when(cid == 0)
    def _():
        pltpu.sync_copy(accb.at[pl.ds(base, RPT)], a_out0.at[pl.ds(base, RPT)])

    @pl.when(cid == 1)
    def _():
        pltpu.sync_copy(accb.at[pl.ds(base, RPT)], a_out1.at[pl.ds(base, RPT)])


def _sc_stage_c(ht, srcC, dstC):
    mesh = plsc.VectorSubcoreMesh(core_axis_name="c", subcore_axis_name="s")
    return pl.kernel(
        _sc_stage_c_body,
        out_type=[
            jax.ShapeDtypeStruct((NP, HID), jnp.bfloat16),
            jax.ShapeDtypeStruct((NP, HID), jnp.bfloat16),
        ],
        mesh=mesh,
        compiler_params=pltpu.CompilerParams(
            needs_layout_passes=False, use_tc_tiling_on_sc=False),
        scratch_types=[
            pltpu.VMEM((BC,), jnp.int32),
            pltpu.VMEM((BC,), jnp.int32),
            pltpu.VMEM((BC,), jnp.int32),
            pltpu.VMEM((BC,), jnp.int32),
            pltpu.VMEM((BC, HID), jnp.bfloat16),
            pltpu.VMEM((BC, HID), jnp.bfloat16),
            pltpu.SemaphoreType.DMA,
            pltpu.SemaphoreType.DMA,
            pltpu.SemaphoreType.DMA,
            pltpu.SemaphoreType.DMA,
            pltpu.VMEM_SHARED((NP, HID), jnp.bfloat16),
        ],
    )(ht, srcC, dstC)


def kernel(x, edge_index, W_gat, att_src, att_dst, b_gat, W1_l, b1_l, W1_r,
           g1, be1, W2_l, b2_l, W2_r, g2, be2, Wc, bc):
    src = edge_index[0]
    dst = edge_index[1]
    loops = jnp.arange(N, dtype=jnp.int32)

    gsrc = jnp.concatenate([src, loops])
    gdst = jnp.concatenate([dst, loops])
    gsrc2 = jnp.pad(gsrc, (0, EGP - EG)).reshape(NSUB * NBG, BG)
    gdst2 = jnp.pad(gdst, (0, EGP - EG), constant_values=N).reshape(NSUB * NBG, BG)
    ssrcA = jnp.pad(src, (0, ESPA - E)).reshape(NSUB * NBSA, BG)
    sdstA = jnp.pad(dst, (0, ESPA - E), constant_values=N).reshape(NSUB * NBSA, BG)
    srcC = jnp.concatenate([
        jnp.pad(src[:EH], (0, EHP - EH)),
        jnp.pad(src[EH:], (0, EHP - EH))]).reshape(2 * NSUB * NBC, BC)
    dstC = jnp.concatenate([
        jnp.pad(dst[:EH], (0, EHP - EH), constant_values=N),
        jnp.pad(dst[EH:], (0, EHP - EH), constant_values=N),
    ]).reshape(2 * NSUB * NBC, BC)

    xP = jnp.pad(x, ((0, NP - N), (0, 0)))

    xw, as2, ad2 = _tc_pre(xP, W_gat,
                           att_src.reshape(H2, 1), att_dst.reshape(H2, 1))

    zpad = jnp.zeros((NP, WI - DI // 2 - 1), jnp.int32)
    asw = _pack_words(jnp.concatenate([as2, jnp.zeros((NP, 1), F32)], axis=1))
    xwt = jnp.concatenate([_pack_words(xw), asw, zpad], axis=1)
    xt = jnp.concatenate(
        [_pack_words(xP), jnp.zeros((NP, WI - DI // 2), jnp.int32)], axis=1)
    md, ag = _sc_stage_a(xwt, xt, ad2.reshape(NP), gsrc2, gdst2, ssrcA, sdstA)

    hlo, hhi, hb = _tc_mid(
        md, ag, xP, b_gat[PERM].reshape(1, H2), W1_l[PERM, :],
        b1_l.reshape(1, H2), W1_r,
        jnp.concatenate([g1[:H2][PERM], g1[H2:]]).reshape(1, HID),
        jnp.concatenate([be1[:H2][PERM], be1[H2:]]).reshape(1, HID))

    W2r_p = jnp.concatenate([W2_r[:H2][PERM], W2_r[H2:]], axis=0)
    hr = _tc_hr(hlo, hhi, W2r_p, b2_l.reshape(1, HID))

    a0, a1 = _sc_stage_c(hb, srcC, dstC)

    W2l_p = jnp.concatenate([W2_l[:H2][PERM], W2_l[H2:]], axis=0)
    deg = lax.slice(ag, (0, DI), (NP, DI + 1))
    out = _tc_fin(a0, a1, deg, hr, W2l_p,
                  g2.reshape(1, HID), be2.reshape(1, HID), Wc,
                  bc.reshape(1, 1))
    return out[:N, 0]


# speedup vs baseline: 1.2801x; 1.0150x over previous
import functools

import numpy as _np

import jax
import jax.numpy as jnp
from jax import lax
from jax.experimental import pallas as pl
from jax.experimental.pallas import tpu as pltpu
from jax.experimental.pallas import tpu_sc as plsc

F32 = jnp.float32

N = 10000
E = 320000
DI = 128
H2 = 128
HID = 256

NSUB = 16
NCORE = 2
BG = 64
BC = 120
RPT = 640
NP = NSUB * RPT

DW = DI + 16
WI = DI // 2 + 8

_base = _np.arange(0, 32, 2)
_blk = _np.concatenate([_base, _base + 1])
PERM = _np.concatenate([_blk + 32 * k for k in range(4)])
PERM2 = PERM[PERM]


def _even_blocks(edges, be):
    nb = -(-edges // (NSUB * be))
    return nb + (nb % 2)


NBG = _even_blocks(E, BG)
NBSA = NBG
EH = E // 2
NBC = -(-EH // (NSUB * BC))
NBC += NBC % 2
ESPA = NSUB * NBSA * BG
EHP = NSUB * NBC * BC

RB = 512
GRID = NP // RB


def _dot(a, b):
    return jnp.dot(a, b, preferred_element_type=F32)


def _tc_pre_body(x_ref, wg_ref, asrc_ref, adst_ref,
                 xw_ref, as_ref, ad_ref):
    x = x_ref[...]
    xw = _dot(x, wg_ref[...])
    xw_ref[...] = xw
    as_ref[...] = _dot(xw, asrc_ref[...])
    ad_ref[...] = _dot(xw, adst_ref[...])


def _tc_pre(xP, W_gat, a_src, a_dst):
    row = lambda i: (i, 0)
    full = lambda i: (0, 0)
    return pl.pallas_call(
        _tc_pre_body,
        grid=(GRID,),
        in_specs=[
            pl.BlockSpec((RB, DI), row),
            pl.BlockSpec((DI, H2), full),
            pl.BlockSpec((H2, 1), full),
            pl.BlockSpec((H2, 1), full),
        ],
        out_specs=[
            pl.BlockSpec((RB, DI), row),
            pl.BlockSpec((RB, 1), row),
            pl.BlockSpec((RB, 1), row),
        ],
        out_shape=[
            jax.ShapeDtypeStruct((NP, DI), F32),
            jax.ShapeDtypeStruct((NP, 1), F32),
            jax.ShapeDtypeStruct((NP, 1), F32),
        ],
    )(xP, W_gat, a_src, a_dst)


def _pack_words(a):
    ab = a.astype(jnp.bfloat16)
    return lax.bitcast_convert_type(
        ab.reshape(a.shape[0], a.shape[1] // 2, 2), jnp.int32)


def _layernorm(h, g, b):
    mu = jnp.mean(h, axis=1, keepdims=True)
    d = h - mu
    var = jnp.mean(d * d, axis=1, keepdims=True)
    return d * lax.rsqrt(var + 1e-5) * g + b


def _tc_mid_body(md_ref, ag_ref, x_ref, xwp_ref, as_ref, ad_ref,
                 bgat_ref, w1l_ref, b1l_ref,
                 w1r_ref, g1_ref, be1_ref,
                 hlo_ref, hhi_ref, hb_ref):
    md = md_ref[...]
    ag = ag_ref[...]
    u = as_ref[...] + ad_ref[...]
    wself = jnp.exp(jnp.maximum(u, 0.2 * u))
    msg = md[:, :H2] + wself * xwp_ref[...]
    den = md[:, H2:H2 + 1] + wself
    x1 = msg / jnp.maximum(den, 1e-16) + bgat_ref[...]
    mean1 = ag[:, :DI] / jnp.maximum(ag[:, DI:DI + 1], 1.0)
    x2 = _dot(mean1, w1l_ref[...]) + b1l_ref[...] + _dot(x_ref[...], w1r_ref[...])
    h = jnp.concatenate([x1, x2], axis=1)
    h = _layernorm(h, g1_ref[...], be1_ref[...])
    h = jnp.maximum(h, 0.0)
    hlo_ref[...] = h[:, :H2]
    hhi_ref[...] = h[:, H2:]
    hb_ref[...] = h.astype(jnp.bfloat16)


def _tc_mid(md, ag, xP, xwp, as2, ad2, bgat, W1_l, b1l, W1_r, g1, be1):
    row = lambda i: (i, 0)
    full = lambda i: (0, 0)
    return pl.pallas_call(
        _tc_mid_body,
        grid=(GRID,),
        in_specs=[
            pl.BlockSpec((RB, DW), row),
            pl.BlockSpec((RB, DW), row),
            pl.BlockSpec((RB, DI), row),
            pl.BlockSpec((RB, H2), row),
            pl.BlockSpec((RB, 1), row),
            pl.BlockSpec((RB, 1), row),
            pl.BlockSpec((1, H2), full),
            pl.BlockSpec((DI, H2), full),
            pl.BlockSpec((1, H2), full),
            pl.BlockSpec((DI, H2), full),
            pl.BlockSpec((1, HID), full),
            pl.BlockSpec((1, HID), full),
        ],
        out_specs=[
            pl.BlockSpec((RB, H2), row),
            pl.BlockSpec((RB, H2), row),
            pl.BlockSpec((RB, HID), row),
        ],
        out_shape=[
            jax.ShapeDtypeStruct((NP, H2), F32),
            jax.ShapeDtypeStruct((NP, H2), F32),
            jax.ShapeDtypeStruct((NP, HID), jnp.bfloat16),
        ],
    )(md, ag, xP, xwp, as2, ad2, bgat, W1_l, b1l, W1_r, g1, be1)


def _tc_fin_body(a0_ref, a1_ref, deg_ref, hlo_ref, hhi_ref,
                 w2l_ref, w2r_ref, b2l_ref,
                 g2_ref, be2_ref, wc_ref, bc_ref, out_ref):
    deg = jnp.maximum(deg_ref[...], 1.0)
    agg2 = a0_ref[...].astype(F32) + a1_ref[...].astype(F32)
    w2r = w2r_ref[...]
    h2 = (_dot(agg2 / deg, w2l_ref[...])
          + _dot(hlo_ref[...], w2r[:H2, :])
          + _dot(hhi_ref[...], w2r[H2:, :]) + b2l_ref[...])
    h2 = _layernorm(h2, g2_ref[...], be2_ref[...])
    h2 = jnp.maximum(h2, 0.0)
    out_ref[...] = _dot(h2, wc_ref[...]) + bc_ref[...]


def _tc_fin(a0, a1, deg, hlo, hhi, W2_l, W2_r, b2l, g2, be2, Wc, bc):
    row = lambda i: (i, 0)
    full = lambda i: (0, 0)
    return pl.pallas_call(
        _tc_fin_body,
        grid=(GRID,),
        in_specs=[
            pl.BlockSpec((RB, HID), row),
            pl.BlockSpec((RB, HID), row),
            pl.BlockSpec((RB, 1), row),
            pl.BlockSpec((RB, H2), row),
            pl.BlockSpec((RB, H2), row),
            pl.BlockSpec((HID, HID), full),
            pl.BlockSpec((HID, HID), full),
            pl.BlockSpec((1, HID), full),
            pl.BlockSpec((1, HID), full),
            pl.BlockSpec((1, HID), full),
            pl.BlockSpec((HID, 1), full),
            pl.BlockSpec((1, 1), full),
        ],
        out_specs=pl.BlockSpec((RB, 1), row),
        out_shape=jax.ShapeDtypeStruct((NP, 1), F32),
    )(a0, a1, deg, hlo, hhi, W2_l, W2_r, b2l, g2, be2, Wc, bc)


def _zero_acc(r0, acc, width, sid):
    zv = jnp.zeros((16,), F32)
    for r in range(16):
        for k in range(width // 16):
            r0[r, pl.ds(k * 16, 16)] = zv
    base = sid * RPT

    def body(i, _):
        off = pl.multiple_of(base + i * 16, 16)
        pltpu.sync_copy(r0.at[pl.ds(0, 16)], acc.at[pl.ds(off, 16)])
        return 0

    lax.fori_loop(0, RPT // 16, body, 0)


def _unpack32(v):
    fa = plsc.bitcast(v << 16, F32)
    fb = plsc.bitcast(v & jnp.int32(-65536), F32)
    return fa, fb


def _pipe_plain(nblk, sid, src2, dst2, table, bufs, acc, width):
    (s0, d0, b0r, f0, g0, ss0), (s1, d1, b1r, f1, g1, ss1) = bufs
    half = nblk // 2
    nb = src2.shape[1]
    wcols = width // 32

    def fire(bid, sidx, didx, gsem, braw):
        pltpu.sync_copy(src2.at[bid], sidx)
        pltpu.sync_copy(dst2.at[bid], didx)
        pltpu.async_copy(table.at[sidx], braw, gsem)

    def process(sidx, didx, gsem, braw, fbuf, ssem):
        pltpu.make_async_copy(table.at[sidx], braw, gsem).wait()
        for j in range(nb):
            for k in range(wcols):
                fa, fb = _unpack32(braw[j, pl.ds(k * 16, 16)])
                fbuf[j, pl.ds(k * 32, 16)] = fa
                fbuf[j, pl.ds(k * 32 + 16, 16)] = fb
        pltpu.async_copy(fbuf, acc.at[didx], ssem, add=True)

    fire(sid * nblk, s0, d0, g0, b0r)
    fire(sid * nblk + 1, s1, d1, g1, b1r)

    def body(i, _):
        bb = sid * nblk + 2 * i
        process(s0, d0, g0, b0r, f0, ss0)
        process(s1, d1, g1, b1r, f1, ss1)

        @pl.when(i < half - 1)
        def _():
            pltpu.make_async_copy(f0, acc.at[d0], ss0).wait()
            fire(bb + 2, s0, d0, g0, b0r)
            pltpu.make_async_copy(f1, acc.at[d1], ss1).wait()
            fire(bb + 3, s1, d1, g1, b1r)

        return 0

    lax.fori_loop(0, half, body, 0)
    pltpu.make_async_copy(f0, acc.at[d0], ss0).wait()
    pltpu.make_async_copy(f1, acc.at[d1], ss1).wait()


def _pipe_gat(sid, src2, dst2, xwt, adb, bufs, acc):
    (s0, d0, b0r, f0, g0, ss0), (s1, d1, b1r, f1, g1, ss1) = bufs
    half = NBG // 2
    lane = lax.iota(jnp.int32, 16)
    as_col = jnp.full((16,), DI // 2, jnp.int32)

    def fire(bid, sidx, didx, gsem, braw):
        pltpu.sync_copy(src2.at[bid], sidx)
        pltpu.sync_copy(dst2.at[bid], didx)
        pltpu.async_copy(xwt.at[sidx], braw, gsem)

    def process(sidx, didx, gsem, braw, fbuf, ssem):
        pltpu.make_async_copy(xwt.at[sidx], braw, gsem).wait()
        for j8 in range(BG // 16):
            jvec = lane + (j8 * 16)
            as_f, _ = _unpack32(plsc.load_gather(braw, [jvec, as_col]))
            ad_f = plsc.load_gather(adb, [didx[pl.ds(j8 * 16, 16)]])
            u = as_f + ad_f
            wv = jnp.exp(jnp.maximum(u, 0.2 * u))
            for l in range(16):
                w = wv[l]
                j = j8 * 16 + l
                for k in range(DI // 32):
                    fa, fb = _unpack32(braw[j, pl.ds(k * 16, 16)])
                    fbuf[j, pl.ds(k * 32, 16)] = fa * w
                    fbuf[j, pl.ds(k * 32 + 16, 16)] = fb * w
                fbuf[j, pl.ds(DI, 16)] = lax.broadcast(w, (16,))
        pltpu.async_copy(fbuf, acc.at[didx], ssem, add=True)

    fire(sid * NBG, s0, d0, g0, b0r)
    fire(sid * NBG + 1, s1, d1, g1, b1r)

    def body(i, _):
        bb = sid * NBG + 2 * i
        process(s0, d0, g0, b0r, f0, ss0)
        process(s1, d1, g1, b1r, f1, ss1)

        @pl.when(i < half - 1)
        def _():
            pltpu.make_async_copy(f0, acc.at[d0], ss0).wait()
            fire(bb + 2, s0, d0, g0, b0r)
            pltpu.make_async_copy(f1, acc.at[d1], ss1).wait()
            fire(bb + 3, s1, d1, g1, b1r)

        return 0

    lax.fori_loop(0, half, body, 0)
    pltpu.make_async_copy(f0, acc.at[d0], ss0).wait()
    pltpu.make_async_copy(f1, acc.at[d1], ss1).wait()


def _sc_stage_a_body(xwt, xt, ad1, gsrc2, gdst2, ssrc2, sdst2,
                     md_out, ag_out,
                     s0, s1, d0, d1, b0r, b1r, adb, f0, f1,
                     g0, g1, ss0, ss1, acc):
    cid = lax.axis_index("c")
    sid = lax.axis_index("s")

    _zero_acc(f0, acc, DW, sid)
    plsc.subcore_barrier()

    @pl.when(cid == 0)
    def _():
        pltpu.sync_copy(ad1, adb)
        _pipe_gat(sid, gsrc2, gdst2, xwt, adb,
                  ((s0, d0, b0r, f0, g0, ss0),
                   (s1, d1, b1r, f1, g1, ss1)),
                  acc)

    @pl.when(cid == 1)
    def _():
        tail = jnp.where(lax.iota(jnp.int32, 16) == 0, 1.0, 0.0)
        for fbuf in (f0, f1):
            for j in range(BG):
                fbuf[j, pl.ds(DI, 16)] = tail
        _pipe_plain(NBSA, sid, ssrc2, sdst2, xt,
                    ((s0, d0, b0r, f0, g0, ss0),
                     (s1, d1, b1r, f1, g1, ss1)), acc, DI)

    plsc.subcore_barrier()

    base = pl.multiple_of(sid * RPT, 16)

    @pl.when(cid == 0)
    def _():
        pltpu.sync_copy(acc.at[pl.ds(base, RPT)], md_out.at[pl.ds(base, RPT)])

    @pl.when(cid == 1)
    def _():
        pltpu.sync_copy(acc.at[pl.ds(base, RPT)], ag_out.at[pl.ds(base, RPT)])


def _sc_stage_a(xwt, xt, ad1, gsrc2, gdst2, ssrc2, sdst2):
    mesh = plsc.VectorSubcoreMesh(core_axis_name="c", subcore_axis_name="s")
    return pl.kernel(
        _sc_stage_a_body,
        out_type=[
            jax.ShapeDtypeStruct((NP, DW), F32),
            jax.ShapeDtypeStruct((NP, DW), F32),
        ],
        mesh=mesh,
        compiler_params=pltpu.CompilerParams(
            needs_layout_passes=False, use_tc_tiling_on_sc=False),
        scratch_types=[
            pltpu.VMEM((BG,), jnp.int32),
            pltpu.VMEM((BG,), jnp.int32),
            pltpu.VMEM((BG,), jnp.int32),
            pltpu.VMEM((BG,), jnp.int32),
            pltpu.VMEM((BG, WI), jnp.int32),
            pltpu.VMEM((BG, WI), jnp.int32),
            pltpu.VMEM((NP,), F32),
            pltpu.VMEM((BG, DW), F32),
            pltpu.VMEM((BG, DW), F32),
            pltpu.SemaphoreType.DMA,
            pltpu.SemaphoreType.DMA,
            pltpu.SemaphoreType.DMA,
            pltpu.SemaphoreType.DMA,
            pltpu.VMEM_SHARED((NP, DW), F32),
        ],
    )(xwt, xt, ad1, gsrc2, gdst2, ssrc2, sdst2)


def _sc_stage_c_body(ht, srcC, dstC, a_out0, a_out1,
                     s0, s1, d0, d1, b0r, b1r, g0, g1, ss0, ss1, accb):
    cid = lax.axis_index("c")
    sid = lax.axis_index("s")

    zv = jnp.zeros((32,), jnp.bfloat16)
    for r in range(16):
        for k in range(HID // 32):
            b0r[r, pl.ds(k * 32, 32)] = zv
    base = pl.multiple_of(sid * RPT, 16)

    def zbody(i, _):
        off = pl.multiple_of(base + i * 16, 16)
        pltpu.sync_copy(b0r.at[pl.ds(0, 16)], accb.at[pl.ds(off, 16)])
        return 0

    lax.fori_loop(0, RPT // 16, zbody, 0)
    plsc.subcore_barrier()

    tbase = (cid * NSUB + sid) * NBC
    half = NBC // 2

    def fire(bid, sidx, didx, gsem, braw):
        pltpu.sync_copy(srcC.at[bid], sidx)
        pltpu.sync_copy(dstC.at[bid], didx)
        pltpu.async_copy(ht.at[sidx], braw, gsem)

    fire(tbase, s0, d0, g0, b0r)
    fire(tbase + 1, s1, d1, g1, b1r)

    def body(i, _):
        bb = tbase + 2 * i
        pltpu.make_async_copy(ht.at[s0], b0r, g0).wait()
        pltpu.async_copy(b0r, accb.at[d0], ss0, add=True)
        pltpu.make_async_copy(ht.at[s1], b1r, g1).wait()
        pltpu.async_copy(b1r, accb.at[d1], ss1, add=True)

        @pl.when(i < half - 1)
        def _():
            pltpu.make_async_copy(b0r, accb.at[d0], ss0).wait()
            fire(bb + 2, s0, d0, g0, b0r)
            pltpu.make_async_copy(b1r, accb.at[d1], ss1).wait()
            fire(bb + 3, s1, d1, g1, b1r)

        return 0

    lax.fori_loop(0, half, body, 0)
    pltpu.make_async_copy(b0r, accb.at[d0], ss0).wait()
    pltpu.make_async_copy(b1r, accb.at[d1], ss1).wait()

    plsc.subcore_barrier()

    @pl.when(cid == 0)
    def _():
        pltpu.sync_copy(accb.at[pl.ds(base, RPT)], a_out0.at[pl.ds(base, RPT)])

    @pl.when(cid == 1)
    def _():
        pltpu.sync_copy(accb.at[pl.ds(base, RPT)], a_out1.at[pl.ds(base, RPT)])


def _sc_stage_c(ht, srcC, dstC):
    mesh = plsc.VectorSubcoreMesh(core_axis_name="c", subcore_axis_name="s")
    return pl.kernel(
        _sc_stage_c_body,
        out_type=[
            jax.ShapeDtypeStruct((NP, HID), jnp.bfloat16),
            jax.ShapeDtypeStruct((NP, HID), jnp.bfloat16),
        ],
        mesh=mesh,
        compiler_params=pltpu.CompilerParams(
            needs_layout_passes=False, use_tc_tiling_on_sc=False),
        scratch_types=[
            pltpu.VMEM((BC,), jnp.int32),
            pltpu.VMEM((BC,), jnp.int32),
            pltpu.VMEM((BC,), jnp.int32),
            pltpu.VMEM((BC,), jnp.int32),
            pltpu.VMEM((BC, HID), jnp.bfloat16),
            pltpu.VMEM((BC, HID), jnp.bfloat16),
            pltpu.SemaphoreType.DMA,
            pltpu.SemaphoreType.DMA,
            pltpu.SemaphoreType.DMA,
            pltpu.SemaphoreType.DMA,
            pltpu.VMEM_SHARED((NP, HID), jnp.bfloat16),
        ],
    )(ht, srcC, dstC)


def kernel(x, edge_index, W_gat, att_src, att_dst, b_gat, W1_l, b1_l, W1_r,
           g1, be1, W2_l, b2_l, W2_r, g2, be2, Wc, bc):
    src = edge_index[0]
    dst = edge_index[1]

    ssrcA = jnp.pad(src, (0, ESPA - E)).reshape(NSUB * NBSA, BG)
    sdstA = jnp.pad(dst, (0, ESPA - E), constant_values=N).reshape(NSUB * NBSA, BG)
    srcC = jnp.concatenate([
        jnp.pad(src[:EH], (0, EHP - EH)),
        jnp.pad(src[EH:], (0, EHP - EH))]).reshape(2 * NSUB * NBC, BC)
    dstC = jnp.concatenate([
        jnp.pad(dst[:EH], (0, EHP - EH), constant_values=N),
        jnp.pad(dst[EH:], (0, EHP - EH), constant_values=N),
    ]).reshape(2 * NSUB * NBC, BC)

    xP = jnp.pad(x, ((0, NP - N), (0, 0)))

    xw, as2, ad2 = _tc_pre(xP, W_gat,
                           att_src.reshape(H2, 1), att_dst.reshape(H2, 1))

    zpad = jnp.zeros((NP, WI - DI // 2 - 1), jnp.int32)
    asw = _pack_words(jnp.concatenate([as2, jnp.zeros((NP, 1), F32)], axis=1))
    xwt = jnp.concatenate([_pack_words(xw), asw, zpad], axis=1)
    xt = jnp.concatenate(
        [_pack_words(xP), jnp.zeros((NP, WI - DI // 2), jnp.int32)], axis=1)
    md, ag = _sc_stage_a(xwt, xt, ad2.reshape(NP), ssrcA, sdstA, ssrcA, sdstA)

    xwp = xw[:, PERM]
    hlo, hhi, hb = _tc_mid(
        md, ag, xP, xwp, as2, ad2, b_gat[PERM].reshape(1, H2), W1_l[PERM, :],
        b1_l.reshape(1, H2), W1_r,
        jnp.concatenate([g1[:H2][PERM], g1[H2:]]).reshape(1, HID),
        jnp.concatenate([be1[:H2][PERM], be1[H2:]]).reshape(1, HID))

    W2r_p = jnp.concatenate([W2_r[:H2][PERM], W2_r[H2:]], axis=0)
    a0, a1 = _sc_stage_c(hb, srcC, dstC)

    W2l_p = jnp.concatenate([W2_l[:H2][PERM], W2_l[H2:]], axis=0)
    deg = lax.slice(ag, (0, DI), (NP, DI + 1))
    out = _tc_fin(a0, a1, deg, hlo, hhi, W2l_p, W2r_p, b2_l.reshape(1, HID),
                  g2.reshape(1, HID), be2.reshape(1, HID), Wc,
                  bc.reshape(1, 1))
    return out[:N, 0]
